# Initial kernel scaffold; baseline (speedup 1.0000x reference)
import jax
import jax.numpy as jnp
from jax.experimental import pallas as pl


def kernel(x, edge_index, edge_attr, batch, Wn, bn, We0, be0, Wl, bl, Wr, br, Wea, att, bias, W1, b1, W2, b2):
    raise NotImplementedError("write your pallas kernel here")


# speedup vs baseline: 1.0035x; 1.0035x over previous
import functools

import jax
import jax.numpy as jnp
from jax.experimental import pallas as pl
from jax.experimental.pallas import tpu as pltpu

N = 10000
E = 320000
F_IN = 128
F_E = 16
H = 256
B = 64
DEPTH = 4

NB = 16
NPAD = 10240
BLK = NPAD // NB


def _pool_ffn_body(h_ref, batch_ref, w1_ref, b1_ref, w2_ref, b2_ref,
                   out_ref, acc_ref, cnt_ref):
    i = pl.program_id(0)

    @pl.when(i == 0)
    def _init():
        acc_ref[...] = jnp.zeros_like(acc_ref)
        cnt_ref[...] = jnp.zeros_like(cnt_ref)

    hb = h_ref[...]
    bb = batch_ref[0, 0, :]
    onehot_t = (bb[None, :] == jax.lax.broadcasted_iota(jnp.int32, (B, BLK), 0)
                ).astype(jnp.float32)
    acc_ref[...] += jnp.dot(onehot_t, hb, preferred_element_type=jnp.float32)
    cnt_ref[...] += jnp.broadcast_to(
        jnp.sum(onehot_t, axis=1, keepdims=True), (B, H))

    @pl.when(i == NB - 1)
    def _fin():
        g = acc_ref[...] / jnp.maximum(cnt_ref[...], 1.0)
        g = jax.nn.relu(jnp.dot(g, w1_ref[...],
                                preferred_element_type=jnp.float32)
                        + b1_ref[...])
        out_ref[...] = jnp.dot(g, w2_ref[...],
                               preferred_element_type=jnp.float32) + b2_ref[...]


def _pool_ffn(h, batch, W1, b1, W2, b2):
    hp = jnp.zeros((NPAD, H), jnp.float32).at[:N].set(h)
    bp = jnp.full((NPAD,), -1, jnp.int32).at[:N].set(batch.astype(jnp.int32))
    bp = bp.reshape(NB, 1, BLK)
    w2p = jnp.zeros((H, 128), jnp.float32).at[:, :1].set(W2)
    b2p = jnp.zeros((1, 128), jnp.float32).at[:, :1].set(b2[None, :])
    out = pl.pallas_call(
        _pool_ffn_body,
        grid=(NB,),
        in_specs=[
            pl.BlockSpec((BLK, H), lambda i: (i, 0)),
            pl.BlockSpec((1, 1, BLK), lambda i: (i, 0, 0)),
            pl.BlockSpec((H, H), lambda i: (0, 0)),
            pl.BlockSpec((1, H), lambda i: (0, 0)),
            pl.BlockSpec((H, 128), lambda i: (0, 0)),
            pl.BlockSpec((1, 128), lambda i: (0, 0)),
        ],
        out_specs=pl.BlockSpec((B, 128), lambda i: (0, 0)),
        out_shape=jax.ShapeDtypeStruct((B, 128), jnp.float32),
        scratch_shapes=[pltpu.VMEM((B, H), jnp.float32),
                        pltpu.VMEM((B, H), jnp.float32)],
    )(hp, bp, W1, b1[None, :], w2p, b2p)
    return out[:, :1]


def _gatv2(x, row, col, ea, Wl, bl, Wr, br, Wea, att, bias):
    n = x.shape[0]
    xl = x @ Wl + bl
    xr = x @ Wr + br
    e = xl[row] + xr[col] + ea @ Wea
    e = jax.nn.leaky_relu(e, 0.2)
    logit = jnp.sum(e * att, axis=-1)
    m = jax.ops.segment_max(logit, col, num_segments=n)
    a = jnp.exp(logit - m[col])
    denom = jax.ops.segment_sum(a, col, num_segments=n)
    alpha = a / jnp.maximum(denom[col], 1e-16)
    out = jax.ops.segment_sum(alpha[:, None] * xl[row], col, num_segments=n)
    return out + bias


def kernel(x, edge_index, edge_attr, batch, Wn, bn, We0, be0, Wl, bl, Wr, br,
           Wea, att, bias, W1, b1, W2, b2):
    h = jax.nn.relu(x @ Wn + bn)
    ea0 = jax.nn.relu(edge_attr @ We0 + be0)
    loop = jnp.arange(N)
    row = jnp.concatenate([edge_index[0], loop])
    col = jnp.concatenate([edge_index[1], loop])
    ea_mean = jnp.mean(ea0, axis=0, keepdims=True)
    ea = jnp.concatenate([ea0, jnp.broadcast_to(ea_mean, (N, H))], axis=0)
    for l in range(DEPTH):
        xh = _gatv2(h, row, col, ea, Wl[l], bl[l], Wr[l], br[l], Wea[l],
                    att[l], bias[l])
        hh = jax.nn.relu(xh) if l < DEPTH - 1 else xh
        h = hh + xh
    return _pool_ffn(h, batch, W1, b1, W2, b2)


# speedup vs baseline: 1.3305x; 1.3258x over previous
import functools

import jax
import jax.numpy as jnp
from jax import lax
from jax.experimental import pallas as pl
from jax.experimental.pallas import tpu as pltpu
from jax.experimental.pallas import tpu_sc as plsc

N = 10000
E = 320000
F_IN = 128
F_E = 16
H = 256
B = 64
DEPTH = 4

EP = E + N
EPP = 331776
C1 = EPP // 32
K = 64
NT = 10240
W = 320
CAP = 11520
DW = 16
SCK = 512
KD = 32
NEG = -3.0e38

NBLK = 1000
EBLK = 512
NEBLK = E // EBLK
NEBLKP = EPP // EBLK


def _tk0_body(x_ref, wn_ref, bn_ref, wl_ref, bl_ref, wr_ref, br_ref,
              h_ref, xl_ref, xr_ref):
    h = jax.nn.relu(jnp.dot(x_ref[...], wn_ref[...],
                            preferred_element_type=jnp.float32) + bn_ref[...])
    h_ref[...] = h
    xl_ref[...] = jnp.dot(h, wl_ref[...],
                          preferred_element_type=jnp.float32) + bl_ref[...]
    xr_ref[...] = jnp.dot(h, wr_ref[...],
                          preferred_element_type=jnp.float32) + br_ref[...]


def _tk0(x, Wn, bn, Wl0, bl0, Wr0, br0):
    g = N // NBLK
    return pl.pallas_call(
        _tk0_body,
        grid=(g,),
        in_specs=[
            pl.BlockSpec((NBLK, F_IN), lambda i: (i, 0)),
            pl.BlockSpec((F_IN, H), lambda i: (0, 0)),
            pl.BlockSpec((1, H), lambda i: (0, 0)),
            pl.BlockSpec((H, H), lambda i: (0, 0)),
            pl.BlockSpec((1, H), lambda i: (0, 0)),
            pl.BlockSpec((H, H), lambda i: (0, 0)),
            pl.BlockSpec((1, H), lambda i: (0, 0)),
        ],
        out_specs=[pl.BlockSpec((NBLK, H), lambda i: (i, 0))] * 3,
        out_shape=[jax.ShapeDtypeStruct((N, H), jnp.float32)] * 3,
    )(x, Wn, bn[None, :], Wl0, bl0[None, :], Wr0, br0[None, :])


def _tkmean_body(ea_ref, we0_ref, be0_ref, out_ref, acc_ref):
    i = pl.program_id(0)

    @pl.when(i == 0)
    def _init():
        acc_ref[...] = jnp.zeros_like(acc_ref)

    ea0 = jax.nn.relu(jnp.dot(ea_ref[...], we0_ref[...],
                              preferred_element_type=jnp.float32) + be0_ref[...])
    acc_ref[...] += jnp.sum(ea0, axis=0, keepdims=True)

    @pl.when(i == NEBLK - 1)
    def _fin():
        out_ref[...] = acc_ref[...] * (1.0 / E)


def _tkmean(edge_attr, We0, be0):
    return pl.pallas_call(
        _tkmean_body,
        grid=(NEBLK,),
        in_specs=[
            pl.BlockSpec((EBLK, F_E), lambda i: (i, 0)),
            pl.BlockSpec((F_E, H), lambda i: (0, 0)),
            pl.BlockSpec((1, H), lambda i: (0, 0)),
        ],
        out_specs=pl.BlockSpec((1, H), lambda i: (0, 0)),
        out_shape=jax.ShapeDtypeStruct((1, H), jnp.float32),
        scratch_shapes=[pltpu.VMEM((1, H), jnp.float32)],
    )(edge_attr, We0, be0[None, :])


def _tkeaw_body(ea_ref, we0_ref, be0_ref, wea_ref, mean_ref, out_ref):
    i = pl.program_id(0)

    @pl.when(i < NEBLK)
    def _real():
        ea0 = jax.nn.relu(jnp.dot(ea_ref[...], we0_ref[...],
                                  preferred_element_type=jnp.float32)
                          + be0_ref[...])
        out_ref[...] = jnp.dot(ea0, wea_ref[...],
                               preferred_element_type=jnp.float32)

    @pl.when(i >= NEBLK)
    def _loops():
        mw = jnp.dot(mean_ref[...], wea_ref[...],
                     preferred_element_type=jnp.float32)
        out_ref[...] = jnp.broadcast_to(mw, (EBLK, H))


def _tkeaw(edge_attr, We0, be0, Wea_l, ea_mean):
    return pl.pallas_call(
        _tkeaw_body,
        grid=(NEBLKP,),
        in_specs=[
            pl.BlockSpec((EBLK, F_E), lambda i: (jnp.minimum(i, NEBLK - 1), 0)),
            pl.BlockSpec((F_E, H), lambda i: (0, 0)),
            pl.BlockSpec((1, H), lambda i: (0, 0)),
            pl.BlockSpec((H, H), lambda i: (0, 0)),
            pl.BlockSpec((1, H), lambda i: (0, 0)),
        ],
        out_specs=pl.BlockSpec((EBLK, H), lambda i: (i, 0)),
        out_shape=jax.ShapeDtypeStruct((EPP, H), jnp.float32),
    )(edge_attr, We0, be0[None, :], Wea_l, ea_mean)


def _tkmred_body(mpart_ref, m_ref):
    m_ref[...] = jnp.max(mpart_ref[...], axis=0, keepdims=True)


def _tkmred(mpart):
    return pl.pallas_call(
        _tkmred_body,
        grid=(1,),
        in_specs=[pl.BlockSpec((32, NT), lambda i: (0, 0))],
        out_specs=pl.BlockSpec((1, NT), lambda i: (0, 0)),
        out_shape=jax.ShapeDtypeStruct((1, NT), jnp.float32),
    )(mpart)


def _tkfin_body(osum_ref, den_ref, bias_ref, wl_ref, bl_ref, wr_ref, br_ref,
                h_ref, xl_ref=None, xr_ref=None, *, last):
    o = osum_ref[...]
    d = den_ref[...]
    xh = o / jnp.maximum(d, 1e-16) + bias_ref[...]
    h = (xh if last else jax.nn.relu(xh)) + xh
    h_ref[...] = h
    if not last:
        xl_ref[...] = jnp.dot(h, wl_ref[...],
                              preferred_element_type=jnp.float32) + bl_ref[...]
        xr_ref[...] = jnp.dot(h, wr_ref[...],
                              preferred_element_type=jnp.float32) + br_ref[...]


def _tkfin(osum, den, bias_l, Wl_n, bl_n, Wr_n, br_n, last):
    g = N // NBLK
    n_out = 1 if last else 3
    out_specs = [pl.BlockSpec((NBLK, H), lambda i: (i, 0))] * n_out
    out_shape = [jax.ShapeDtypeStruct((N, H), jnp.float32)] * n_out
    out = pl.pallas_call(
        functools.partial(_tkfin_body, last=last),
        grid=(g,),
        in_specs=[
            pl.BlockSpec((NBLK, H), lambda i: (i, 0)),
            pl.BlockSpec((NBLK, 1), lambda i: (i, 0)),
            pl.BlockSpec((1, H), lambda i: (0, 0)),
            pl.BlockSpec((H, H), lambda i: (0, 0)),
            pl.BlockSpec((1, H), lambda i: (0, 0)),
            pl.BlockSpec((H, H), lambda i: (0, 0)),
            pl.BlockSpec((1, H), lambda i: (0, 0)),
        ],
        out_specs=out_specs,
        out_shape=out_shape,
    )(osum, den, bias_l[None, :], Wl_n, bl_n[None, :], Wr_n, br_n[None, :])
    if last:
        return out[0] if isinstance(out, (list, tuple)) else out
    return out


def _gather16(vals, idx):
    return lax.gather(
        vals, idx[:, None],
        lax.GatherDimensionNumbers(offset_dims=(), collapsed_slice_dims=(0,),
                                   start_index_map=(0,)),
        (1,), mode=lax.GatherScatterMode.PROMISE_IN_BOUNDS)


def _iota16():
    return lax.broadcasted_iota(jnp.int32, (16,), 0)


def _lane_sum_all(v):
    for d in (1, 2, 4, 8):
        v = v + _gather16(v, jnp.bitwise_xor(_iota16(), d))
    return v


def _seg_scatter_max(table, col16, val16):
    iota = _iota16()
    fill = jnp.float32(NEG)
    acc = val16
    any_earlier = iota < 0
    for j in range(1, 16):
        idx = jnp.bitwise_and(iota - j, 15)
        rot_c = _gather16(col16, idx)
        rot_v = _gather16(val16, idx)
        same = rot_c == col16
        acc = jnp.maximum(acc, jnp.where(same, rot_v, fill))
        any_earlier = any_earlier | (same & (iota >= j))
    first = jnp.logical_not(any_earlier)
    cur = plsc.load_gather(table, [col16])
    plsc.store_scatter(table, [col16], jnp.maximum(cur, acc), mask=first)


def _sk1_body(row_h, col_h, xl_h, xr_h, eaw_h, att_h,
              logit_h, mpart_h,
              ridx, cidx, xlg, xrg, eawv, attv, lstage, mv, sem1, sem2):
    c = lax.axis_index("c")
    s = lax.axis_index("s")
    wid = s * 2 + c
    e0 = wid * C1
    pltpu.sync_copy(att_h, attv)

    @pl.loop(0, NT // 16)
    def _initm(i):
        mv[pl.ds(i * 16, 16)] = jnp.full((16,), NEG, jnp.float32)

    @pl.loop(0, C1 // K)
    def _chunk(sc):
        base = e0 + sc * K
        pltpu.sync_copy(row_h.at[pl.ds(base, K)], ridx)
        pltpu.sync_copy(col_h.at[pl.ds(base, K)], cidx)
        d1 = pltpu.async_copy(xl_h.at[ridx], xlg, sem1)
        d2 = pltpu.async_copy(xr_h.at[cidx], xrg, sem2)
        pltpu.sync_copy(eaw_h.at[pl.ds(base, K)], eawv)
        d1.wait()
        d2.wait()

        @pl.loop(0, K // 16)
        def _q(q):
            lane = _iota16()
            logit16 = jnp.zeros((16,), jnp.float32)
            for j in range(16):
                jj = q * 16 + j
                acc = jnp.zeros((16,), jnp.float32)
                for v in range(16):
                    sl = pl.ds(v * 16, 16)
                    u = xlg[jj, sl] + xrg[jj, sl] + eawv[jj, sl]
                    u = jnp.maximum(u, 0.2 * u)
                    acc = acc + u * attv[sl]
                tot = _lane_sum_all(acc)
                logit16 = jnp.where(lane == j, tot, logit16)
            slq = pl.ds(q * 16, 16)
            lstage[slq] = logit16
            col16 = cidx[slq]
            _seg_scatter_max(mv, col16, logit16)

        pltpu.sync_copy(lstage, logit_h.at[pl.ds(base, K)])

    pltpu.sync_copy(mv, mpart_h.at[wid])


def _sk1(row, col, xl, xr, eaw, att_l):
    fn = pl.kernel(
        _sk1_body,
        out_type=(jax.ShapeDtypeStruct((EPP,), jnp.float32),
                  jax.ShapeDtypeStruct((32, NT), jnp.float32)),
        mesh=plsc.VectorSubcoreMesh(core_axis_name="c", subcore_axis_name="s"),
        scratch_types=(
            pltpu.VMEM((K,), jnp.int32),
            pltpu.VMEM((K,), jnp.int32),
            pltpu.VMEM((K, H), jnp.float32),
            pltpu.VMEM((K, H), jnp.float32),
            pltpu.VMEM((K, H), jnp.float32),
            pltpu.VMEM((H,), jnp.float32),
            pltpu.VMEM((K,), jnp.float32),
            pltpu.VMEM((NT,), jnp.float32),
            pltpu.SemaphoreType.DMA,
            pltpu.SemaphoreType.DMA,
        ),
        compiler_params=pltpu.CompilerParams(needs_layout_passes=False),
    )
    return fn(row, col, xl, xr, eaw, att_l)


def _sk2_body(row_h, col_h, logit_h, xl_h, m_h,
              osum_h, den_h,
              rowb, colb, lb, packed, acomp, ridx, rows, mv, win, dwin, sem1):
    c = lax.axis_index("c")
    s = lax.axis_index("s")
    wid = s * 2 + c
    base = wid * W

    pltpu.sync_copy(m_h.at[0], mv)

    @pl.loop(0, CAP // 16)
    def _zc(i):
        sl = pl.ds(i * 16, 16)
        packed[sl] = jnp.zeros((16,), jnp.int32)
        acomp[sl] = jnp.zeros((16,), jnp.float32)

    @pl.loop(0, W)
    def _zw(r):
        for v in range(H // 16):
            win[r, pl.ds(v * 16, 16)] = jnp.zeros((16,), jnp.float32)

    @pl.loop(0, 3)
    def _zd(r):
        for v in range(8):
            dwin[r, pl.ds(v * 16, 16)] = jnp.zeros((16,), jnp.float32)

    def _scan_chunk(i, off):
        b0 = i * SCK
        pltpu.sync_copy(row_h.at[pl.ds(b0, SCK)], rowb)
        pltpu.sync_copy(col_h.at[pl.ds(b0, SCK)], colb)
        pltpu.sync_copy(logit_h.at[pl.ds(b0, SCK)], lb)
        for q in range(SCK // 16):
            sl = pl.ds(q * 16, 16)
            col16 = colb[sl]
            row16 = rowb[sl]
            l16 = lb[sl]
            m16 = plsc.load_gather(mv, [col16])
            a16 = jnp.exp(l16 - m16)
            mask = (col16 >= base) & (col16 < base + W)
            packed16 = row16 * 512 + (col16 - base)
            plsc.store_compressed(packed.at[pl.ds(off, 16)], packed16,
                                  mask=mask)
            plsc.store_compressed(acomp.at[pl.ds(off, 16)], a16, mask=mask)
            cnt = plsc.all_reduce_population_count(mask)[0]
            off = jnp.minimum(off + cnt, CAP - 2 * KD)
        return off

    nh = pl.loop(0, EPP // SCK, init_carry=jnp.int32(0))(_scan_chunk)

    ndr = (nh + KD - 1) // KD

    @pl.loop(0, ndr)
    def _drain(t):
        j0 = t * KD
        for q in range(KD // 16):
            sl = pl.ds(j0 + q * 16, 16)
            ridx[pl.ds(q * 16, 16)] = jnp.right_shift(packed[sl], 9)
        pltpu.async_copy(xl_h.at[ridx], rows, sem1).wait()
        for q in range(KD // 16):
            sl = pl.ds(j0 + q * 16, 16)
            cl16 = jnp.bitwise_and(packed[sl], 511)
            a16 = acomp[sl]
            lane = _iota16()
            for j in range(16):
                aj = a16[j]
                cj = cl16[j]
                jj = q * 16 + j
                for v in range(H // 16):
                    slv = pl.ds(v * 16, 16)
                    win[cj, slv] = win[cj, slv] + rows[jj, slv] * aj
                cjr = jnp.right_shift(cj, 7)
                cjc = jnp.bitwise_and(cj, 127)
                cjc0 = cjc - jnp.bitwise_and(cjc, 15)
                lt = jnp.bitwise_and(cjc, 15)
                sld = pl.ds(cjc0, 16)
                dwin[cjr, sld] = dwin[cjr, sld] + jnp.where(lane == lt, aj,
                                                            0.0)

    pltpu.sync_copy(win, osum_h.at[pl.ds(base, W)])
    pltpu.sync_copy(dwin, den_h.at[wid])


def _sk2(row, col, logit, xl, m):
    fn = pl.kernel(
        _sk2_body,
        out_type=(jax.ShapeDtypeStruct((NT, H), jnp.float32),
                  jax.ShapeDtypeStruct((32, 3, 128), jnp.float32)),
        mesh=plsc.VectorSubcoreMesh(core_axis_name="c", subcore_axis_name="s"),
        scratch_types=(
            pltpu.VMEM((SCK,), jnp.int32),
            pltpu.VMEM((SCK,), jnp.int32),
            pltpu.VMEM((SCK,), jnp.float32),
            pltpu.VMEM((CAP,), jnp.int32),
            pltpu.VMEM((CAP,), jnp.float32),
            pltpu.VMEM((KD,), jnp.int32),
            pltpu.VMEM((KD, H), jnp.float32),
            pltpu.VMEM((NT,), jnp.float32),
            pltpu.VMEM((W, H), jnp.float32),
            pltpu.VMEM((3, 128), jnp.float32),
            pltpu.SemaphoreType.DMA,
        ),
        compiler_params=pltpu.CompilerParams(needs_layout_passes=False),
    )
    return fn(row, col, logit, xl, m)


NB = 16
NPAD = 10240
BLK = NPAD // NB


def _pool_ffn_body(h_ref, batch_ref, w1_ref, b1_ref, w2_ref, b2_ref,
                   out_ref, acc_ref, cnt_ref):
    i = pl.program_id(0)

    @pl.when(i == 0)
    def _init():
        acc_ref[...] = jnp.zeros_like(acc_ref)
        cnt_ref[...] = jnp.zeros_like(cnt_ref)

    hb = h_ref[...]
    bb = batch_ref[0, 0, :]
    onehot_t = (bb[None, :] == jax.lax.broadcasted_iota(jnp.int32, (B, BLK), 0)
                ).astype(jnp.float32)
    acc_ref[...] += jnp.dot(onehot_t, hb, preferred_element_type=jnp.float32)
    cnt_ref[...] += jnp.broadcast_to(
        jnp.sum(onehot_t, axis=1, keepdims=True), (B, H))

    @pl.when(i == NB - 1)
    def _fin():
        g = acc_ref[...] / jnp.maximum(cnt_ref[...], 1.0)
        g = jax.nn.relu(jnp.dot(g, w1_ref[...],
                                preferred_element_type=jnp.float32)
                        + b1_ref[...])
        out_ref[...] = jnp.dot(g, w2_ref[...],
                               preferred_element_type=jnp.float32) + b2_ref[...]


def _pool_ffn(h, batch, W1, b1, W2, b2):
    hp = jnp.zeros((NPAD, H), jnp.float32).at[:N].set(h)
    bp = jnp.full((NPAD,), -1, jnp.int32).at[:N].set(batch.astype(jnp.int32))
    bp = bp.reshape(NB, 1, BLK)
    w2p = jnp.zeros((H, 128), jnp.float32).at[:, :1].set(W2)
    b2p = jnp.zeros((1, 128), jnp.float32).at[:, :1].set(b2[None, :])
    out = pl.pallas_call(
        _pool_ffn_body,
        grid=(NB,),
        in_specs=[
            pl.BlockSpec((BLK, H), lambda i: (i, 0)),
            pl.BlockSpec((1, 1, BLK), lambda i: (i, 0, 0)),
            pl.BlockSpec((H, H), lambda i: (0, 0)),
            pl.BlockSpec((1, H), lambda i: (0, 0)),
            pl.BlockSpec((H, 128), lambda i: (0, 0)),
            pl.BlockSpec((1, 128), lambda i: (0, 0)),
        ],
        out_specs=pl.BlockSpec((B, 128), lambda i: (0, 0)),
        out_shape=jax.ShapeDtypeStruct((B, 128), jnp.float32),
        scratch_shapes=[pltpu.VMEM((B, H), jnp.float32),
                        pltpu.VMEM((B, H), jnp.float32)],
    )(hp, bp, W1, b1[None, :], w2p, b2p)
    return out[:, :1]


def kernel(x, edge_index, edge_attr, batch, Wn, bn, We0, be0, Wl, bl, Wr, br,
           Wea, att, bias, W1, b1, W2, b2):
    ei = edge_index.astype(jnp.int32)
    loop = jnp.arange(N, dtype=jnp.int32)
    row = jnp.concatenate([ei[0], loop,
                           jnp.zeros((EPP - EP,), jnp.int32)])
    col = jnp.concatenate([ei[1], loop,
                           jnp.full((EPP - EP,), N, jnp.int32)])

    h, xl, xr = _tk0(x, Wn, bn, Wl[0], bl[0], Wr[0], br[0])
    ea_mean = _tkmean(edge_attr, We0, be0)

    for l in range(DEPTH):
        eaw = _tkeaw(edge_attr, We0, be0, Wea[l], ea_mean)
        logit, mpart = _sk1(row, col, xl, xr, eaw, att[l])
        m = _tkmred(mpart)
        osum, den = _sk2(row, col, logit, xl, m)
        den = den.reshape(32, 384)[:, :W].reshape(NT, 1)
        last = l == DEPTH - 1
        if last:
            h = _tkfin(osum, den, bias[l], Wl[0], bl[0], Wr[0], br[0], True)
        else:
            h, xl, xr = _tkfin(osum, den, bias[l], Wl[l + 1], bl[l + 1],
                               Wr[l + 1], br[l + 1], False)

    return _pool_ffn(h, batch, W1, b1, W2, b2)


# speedup vs baseline: 1.6125x; 1.2120x over previous
import functools

import jax
import jax.numpy as jnp
from jax import lax
from jax.experimental import pallas as pl
from jax.experimental.pallas import tpu as pltpu
from jax.experimental.pallas import tpu_sc as plsc

N = 10000
E = 320000
F_IN = 128
F_E = 16
H = 256
B = 64
DEPTH = 4

EP = E + N
EPP = 331776
C1 = EPP // 32
K = 64
NT = 10240
W = 320
CAP = 11520
DW = 16
SCK = 2048
KD = 32
NEG = -3.0e38

NBLK = 1000
EBLK = 512
NEBLK = E // EBLK
NEBLKP = EPP // EBLK


def _tk0_body(x_ref, wn_ref, bn_ref, wl_ref, bl_ref, wr_ref, br_ref,
              h_ref, xl_ref, xr_ref):
    h = jax.nn.relu(jnp.dot(x_ref[...], wn_ref[...],
                            preferred_element_type=jnp.float32) + bn_ref[...])
    h_ref[...] = h
    xl_ref[...] = jnp.dot(h, wl_ref[...],
                          preferred_element_type=jnp.float32) + bl_ref[...]
    xr_ref[...] = jnp.dot(h, wr_ref[...],
                          preferred_element_type=jnp.float32) + br_ref[...]


def _tk0(x, Wn, bn, Wl0, bl0, Wr0, br0):
    g = N // NBLK
    return pl.pallas_call(
        _tk0_body,
        grid=(g,),
        in_specs=[
            pl.BlockSpec((NBLK, F_IN), lambda i: (i, 0)),
            pl.BlockSpec((F_IN, H), lambda i: (0, 0)),
            pl.BlockSpec((1, H), lambda i: (0, 0)),
            pl.BlockSpec((H, H), lambda i: (0, 0)),
            pl.BlockSpec((1, H), lambda i: (0, 0)),
            pl.BlockSpec((H, H), lambda i: (0, 0)),
            pl.BlockSpec((1, H), lambda i: (0, 0)),
        ],
        out_specs=[pl.BlockSpec((NBLK, H), lambda i: (i, 0))] * 3,
        out_shape=[jax.ShapeDtypeStruct((N, H), jnp.float32)] * 3,
    )(x, Wn, bn[None, :], Wl0, bl0[None, :], Wr0, br0[None, :])


def _tkmean_body(ea_ref, we0_ref, be0_ref, out_ref, acc_ref):
    i = pl.program_id(0)

    @pl.when(i == 0)
    def _init():
        acc_ref[...] = jnp.zeros_like(acc_ref)

    ea0 = jax.nn.relu(jnp.dot(ea_ref[...], we0_ref[...],
                              preferred_element_type=jnp.float32) + be0_ref[...])
    acc_ref[...] += jnp.sum(ea0, axis=0, keepdims=True)

    @pl.when(i == NEBLK - 1)
    def _fin():
        out_ref[...] = acc_ref[...] * (1.0 / E)


def _tkmean(edge_attr, We0, be0):
    return pl.pallas_call(
        _tkmean_body,
        grid=(NEBLK,),
        in_specs=[
            pl.BlockSpec((EBLK, F_E), lambda i: (i, 0)),
            pl.BlockSpec((F_E, H), lambda i: (0, 0)),
            pl.BlockSpec((1, H), lambda i: (0, 0)),
        ],
        out_specs=pl.BlockSpec((1, H), lambda i: (0, 0)),
        out_shape=jax.ShapeDtypeStruct((1, H), jnp.float32),
        scratch_shapes=[pltpu.VMEM((1, H), jnp.float32)],
    )(edge_attr, We0, be0[None, :])


def _tkeaw_body(ea_ref, we0_ref, be0_ref, wea_ref, mean_ref, out_ref):
    i = pl.program_id(0)

    @pl.when(i < NEBLK)
    def _real():
        ea0 = jax.nn.relu(jnp.dot(ea_ref[...], we0_ref[...],
                                  preferred_element_type=jnp.float32)
                          + be0_ref[...])
        out_ref[...] = jnp.dot(ea0, wea_ref[...],
                               preferred_element_type=jnp.float32)

    @pl.when(i >= NEBLK)
    def _loops():
        mw = jnp.dot(mean_ref[...], wea_ref[...],
                     preferred_element_type=jnp.float32)
        out_ref[...] = jnp.broadcast_to(mw, (EBLK, H))


def _tkeaw(edge_attr, We0, be0, Wea_l, ea_mean):
    return pl.pallas_call(
        _tkeaw_body,
        grid=(NEBLKP,),
        in_specs=[
            pl.BlockSpec((EBLK, F_E), lambda i: (jnp.minimum(i, NEBLK - 1), 0)),
            pl.BlockSpec((F_E, H), lambda i: (0, 0)),
            pl.BlockSpec((1, H), lambda i: (0, 0)),
            pl.BlockSpec((H, H), lambda i: (0, 0)),
            pl.BlockSpec((1, H), lambda i: (0, 0)),
        ],
        out_specs=pl.BlockSpec((EBLK, H), lambda i: (i, 0)),
        out_shape=jax.ShapeDtypeStruct((EPP, H), jnp.float32),
    )(edge_attr, We0, be0[None, :], Wea_l, ea_mean)


def _tkmred_body(mpart_ref, m_ref):
    m_ref[...] = jnp.max(mpart_ref[...], axis=0, keepdims=True)


def _tkmred(mpart):
    return pl.pallas_call(
        _tkmred_body,
        grid=(1,),
        in_specs=[pl.BlockSpec((32, NT), lambda i: (0, 0))],
        out_specs=pl.BlockSpec((1, NT), lambda i: (0, 0)),
        out_shape=jax.ShapeDtypeStruct((1, NT), jnp.float32),
    )(mpart)


def _tkfin_body(osum_ref, den_ref, bias_ref, wl_ref, bl_ref, wr_ref, br_ref,
                h_ref, xl_ref=None, xr_ref=None, *, last):
    o = osum_ref[...]
    d = den_ref[...]
    xh = o / jnp.maximum(d, 1e-16) + bias_ref[...]
    h = (xh if last else jax.nn.relu(xh)) + xh
    h_ref[...] = h
    if not last:
        xl_ref[...] = jnp.dot(h, wl_ref[...],
                              preferred_element_type=jnp.float32) + bl_ref[...]
        xr_ref[...] = jnp.dot(h, wr_ref[...],
                              preferred_element_type=jnp.float32) + br_ref[...]


def _tkfin(osum, den, bias_l, Wl_n, bl_n, Wr_n, br_n, last):
    g = N // NBLK
    n_out = 1 if last else 3
    out_specs = [pl.BlockSpec((NBLK, H), lambda i: (i, 0))] * n_out
    out_shape = [jax.ShapeDtypeStruct((N, H), jnp.float32)] * n_out
    out = pl.pallas_call(
        functools.partial(_tkfin_body, last=last),
        grid=(g,),
        in_specs=[
            pl.BlockSpec((NBLK, H), lambda i: (i, 0)),
            pl.BlockSpec((NBLK, 1), lambda i: (i, 0)),
            pl.BlockSpec((1, H), lambda i: (0, 0)),
            pl.BlockSpec((H, H), lambda i: (0, 0)),
            pl.BlockSpec((1, H), lambda i: (0, 0)),
            pl.BlockSpec((H, H), lambda i: (0, 0)),
            pl.BlockSpec((1, H), lambda i: (0, 0)),
        ],
        out_specs=out_specs,
        out_shape=out_shape,
    )(osum, den, bias_l[None, :], Wl_n, bl_n[None, :], Wr_n, br_n[None, :])
    if last:
        return out[0] if isinstance(out, (list, tuple)) else out
    return out


def _gather16(vals, idx):
    return lax.gather(
        vals, idx[:, None],
        lax.GatherDimensionNumbers(offset_dims=(), collapsed_slice_dims=(0,),
                                   start_index_map=(0,)),
        (1,), mode=lax.GatherScatterMode.PROMISE_IN_BOUNDS)


def _iota16():
    return lax.broadcasted_iota(jnp.int32, (16,), 0)


def _lane_sum_all(v):
    for d in (1, 2, 4, 8):
        v = v + _gather16(v, jnp.bitwise_xor(_iota16(), d))
    return v


def _seg_scatter_max(table, col16, val16):
    iota = _iota16()
    fill = jnp.float32(NEG)
    acc = val16
    any_earlier = iota < 0
    for j in range(1, 16):
        idx = jnp.bitwise_and(iota - j, 15)
        rot_c = _gather16(col16, idx)
        rot_v = _gather16(val16, idx)
        same = rot_c == col16
        acc = jnp.maximum(acc, jnp.where(same, rot_v, fill))
        any_earlier = any_earlier | (same & (iota >= j))
    first = jnp.logical_not(any_earlier)
    cur = plsc.load_gather(table, [col16])
    plsc.store_scatter(table, [col16], jnp.maximum(cur, acc), mask=first)


def _sk1_body(row_h, col_h, xl_h, xr_h, eaw_h, att_h,
              logit_h, mpart_h,
              ridx, cidx, xlg, xrg, eawv, attv, lstage, mv, sem1, sem2):
    c = lax.axis_index("c")
    s = lax.axis_index("s")
    wid = s * 2 + c
    e0 = wid * C1
    pltpu.sync_copy(att_h, attv)

    @pl.loop(0, NT // 16)
    def _initm(i):
        mv[pl.ds(i * 16, 16)] = jnp.full((16,), NEG, jnp.float32)

    @pl.loop(0, C1 // K)
    def _chunk(sc):
        base = e0 + sc * K
        pltpu.sync_copy(row_h.at[pl.ds(base, K)], ridx)
        pltpu.sync_copy(col_h.at[pl.ds(base, K)], cidx)
        d1 = pltpu.async_copy(xl_h.at[ridx], xlg, sem1)
        d2 = pltpu.async_copy(xr_h.at[cidx], xrg, sem2)
        pltpu.sync_copy(eaw_h.at[pl.ds(base, K)], eawv)
        d1.wait()
        d2.wait()

        @pl.loop(0, K // 16)
        def _q(q):
            lane = _iota16()
            logit16 = jnp.zeros((16,), jnp.float32)
            for j in range(16):
                jj = q * 16 + j
                acc = jnp.zeros((16,), jnp.float32)
                for v in range(16):
                    sl = pl.ds(v * 16, 16)
                    u = xlg[jj, sl] + xrg[jj, sl] + eawv[jj, sl]
                    u = jnp.maximum(u, 0.2 * u)
                    acc = acc + u * attv[sl]
                tot = _lane_sum_all(acc)
                logit16 = jnp.where(lane == j, tot, logit16)
            slq = pl.ds(q * 16, 16)
            lstage[slq] = logit16
            col16 = cidx[slq]
            _seg_scatter_max(mv, col16, logit16)

        pltpu.sync_copy(lstage, logit_h.at[pl.ds(base, K)])

    pltpu.sync_copy(mv, mpart_h.at[wid])


def _sk1(row, col, xl, xr, eaw, att_l):
    fn = pl.kernel(
        _sk1_body,
        out_type=(jax.ShapeDtypeStruct((EPP,), jnp.float32),
                  jax.ShapeDtypeStruct((32, NT), jnp.float32)),
        mesh=plsc.VectorSubcoreMesh(core_axis_name="c", subcore_axis_name="s"),
        scratch_types=(
            pltpu.VMEM((K,), jnp.int32),
            pltpu.VMEM((K,), jnp.int32),
            pltpu.VMEM((K, H), jnp.float32),
            pltpu.VMEM((K, H), jnp.float32),
            pltpu.VMEM((K, H), jnp.float32),
            pltpu.VMEM((H,), jnp.float32),
            pltpu.VMEM((K,), jnp.float32),
            pltpu.VMEM((NT,), jnp.float32),
            pltpu.SemaphoreType.DMA,
            pltpu.SemaphoreType.DMA,
        ),
        compiler_params=pltpu.CompilerParams(needs_layout_passes=False),
    )
    return fn(row, col, xl, xr, eaw, att_l)


def _sk2a_body(row_h, col_h, logit_h, m_h,
               pc_h, ac_h, cnt_h,
               rowb0, colb0, lb0, rowb1, colb1, lb1,
               packed, acomp, mv, stg, sem0, sem1):
    c = lax.axis_index("c")
    s = lax.axis_index("s")
    wid = s * 2 + c
    base = wid * W

    pltpu.sync_copy(m_h.at[0], mv)

    @pl.loop(0, CAP // 16)
    def _zc(i):
        sl = pl.ds(i * 16, 16)
        packed[sl] = jnp.zeros((16,), jnp.int32)
        acomp[sl] = jnp.zeros((16,), jnp.float32)

    nchunks = EPP // SCK

    def _issue(ci, bufs, sem):
        b0 = ci * SCK
        pltpu.async_copy(row_h.at[pl.ds(b0, SCK)], bufs[0], sem)
        pltpu.async_copy(col_h.at[pl.ds(b0, SCK)], bufs[1], sem)
        pltpu.async_copy(logit_h.at[pl.ds(b0, SCK)], bufs[2], sem)

    def _wait(ci, bufs, sem):
        b0 = ci * SCK
        pltpu.make_async_copy(row_h.at[pl.ds(b0, SCK)], bufs[0], sem).wait()
        pltpu.make_async_copy(col_h.at[pl.ds(b0, SCK)], bufs[1], sem).wait()
        pltpu.make_async_copy(logit_h.at[pl.ds(b0, SCK)], bufs[2], sem).wait()

    def _scan(bufs, off):
        rowb, colb, lb = bufs
        for q in range(SCK // 16):
            sl = pl.ds(q * 16, 16)
            col16 = colb[sl]
            row16 = rowb[sl]
            l16 = lb[sl]
            m16 = plsc.load_gather(mv, [col16])
            a16 = jnp.exp(l16 - m16)
            mask = (col16 >= base) & (col16 < base + W)
            packed16 = row16 * 512 + (col16 - base)
            plsc.store_compressed(packed.at[pl.ds(off, 16)], packed16,
                                  mask=mask)
            plsc.store_compressed(acomp.at[pl.ds(off, 16)], a16, mask=mask)
            cnt = plsc.all_reduce_population_count(mask)[0]
            off = jnp.minimum(off + cnt, CAP - 2 * KD)
        return off

    bufs0 = (rowb0, colb0, lb0)
    bufs1 = (rowb1, colb1, lb1)
    _issue(0, bufs0, sem0)

    def _outer(i, off):
        c0 = 2 * i
        _issue(jnp.minimum(c0 + 1, nchunks - 1), bufs1, sem1)
        _wait(c0, bufs0, sem0)
        off = _scan(bufs0, off)
        _issue(jnp.minimum(c0 + 2, nchunks - 1), bufs0, sem0)
        _wait(jnp.minimum(c0 + 1, nchunks - 1), bufs1, sem1)
        off = _scan(bufs1, off)
        return off

    nh = pl.loop(0, nchunks // 2, init_carry=jnp.int32(0))(_outer)
    _wait(nchunks - 1, bufs0, sem0)

    lane = _iota16()
    stg[pl.ds(0, 16)] = jnp.where(lane == 0, nh, 0)
    pltpu.sync_copy(stg, cnt_h.at[wid])
    pltpu.sync_copy(packed, pc_h.at[wid])
    pltpu.sync_copy(acomp, ac_h.at[wid])


def _sk2a(row, col, logit, m):
    fn = pl.kernel(
        _sk2a_body,
        out_type=(jax.ShapeDtypeStruct((32, CAP), jnp.int32),
                  jax.ShapeDtypeStruct((32, CAP), jnp.float32),
                  jax.ShapeDtypeStruct((32, 16), jnp.int32)),
        mesh=plsc.VectorSubcoreMesh(core_axis_name="c", subcore_axis_name="s"),
        scratch_types=(
            pltpu.VMEM((SCK,), jnp.int32),
            pltpu.VMEM((SCK,), jnp.int32),
            pltpu.VMEM((SCK,), jnp.float32),
            pltpu.VMEM((SCK,), jnp.int32),
            pltpu.VMEM((SCK,), jnp.int32),
            pltpu.VMEM((SCK,), jnp.float32),
            pltpu.VMEM((CAP,), jnp.int32),
            pltpu.VMEM((CAP,), jnp.float32),
            pltpu.VMEM((NT,), jnp.float32),
            pltpu.VMEM((16,), jnp.int32),
            pltpu.SemaphoreType.DMA,
            pltpu.SemaphoreType.DMA,
        ),
        compiler_params=pltpu.CompilerParams(needs_layout_passes=False),
    )
    return fn(row, col, logit, m)


def _sk2b_body(xl_h, pc_h, ac_h, cnt_h,
               osum_h, den_h,
               packed, acomp, ridx0, ridx1, rows0, rows1, win, dwin, stg,
               sem0, sem1):
    c = lax.axis_index("c")
    s = lax.axis_index("s")
    wid = s * 2 + c
    base = wid * W

    pltpu.sync_copy(cnt_h.at[wid], stg)
    nh = stg[pl.ds(0, 16)][0]
    pltpu.sync_copy(pc_h.at[wid], packed)
    pltpu.sync_copy(ac_h.at[wid], acomp)

    @pl.loop(0, W)
    def _zw(r):
        for v in range(H // 16):
            win[r, pl.ds(v * 16, 16)] = jnp.zeros((16,), jnp.float32)

    @pl.loop(0, 3)
    def _zd(r):
        for v in range(8):
            dwin[r, pl.ds(v * 16, 16)] = jnp.zeros((16,), jnp.float32)

    ndr = (nh + KD - 1) // KD
    maxj0 = CAP - KD

    def _issue(t, ridx, rows, sem):
        j0 = jnp.minimum(t * KD, maxj0)
        for q in range(KD // 16):
            sl = pl.ds(j0 + q * 16, 16)
            ridx[pl.ds(q * 16, 16)] = jnp.right_shift(packed[sl], 9)
        pltpu.async_copy(xl_h.at[ridx], rows, sem)

    def _wait(ridx, rows, sem):
        pltpu.make_async_copy(xl_h.at[ridx], rows, sem).wait()

    def _acc(t, rows):
        j0 = jnp.minimum(t * KD, maxj0)
        lane = _iota16()
        for q in range(KD // 16):
            sl = pl.ds(j0 + q * 16, 16)
            cl16 = jnp.bitwise_and(packed[sl], 511)
            a16 = acomp[sl]
            for j in range(16):
                aj = a16[j]
                cj = cl16[j]
                jj = q * 16 + j
                for v in range(H // 16):
                    slv = pl.ds(v * 16, 16)
                    win[cj, slv] = win[cj, slv] + rows[jj, slv] * aj
                cjr = jnp.right_shift(cj, 7)
                cjc = jnp.bitwise_and(cj, 127)
                cjc0 = cjc - jnp.bitwise_and(cjc, 15)
                lt = jnp.bitwise_and(cjc, 15)
                sld = pl.ds(cjc0, 16)
                dwin[cjr, sld] = dwin[cjr, sld] + jnp.where(lane == lt, aj,
                                                            0.0)

    nouter = (ndr + 1) // 2
    _issue(0, ridx0, rows0, sem0)

    @pl.loop(0, nouter)
    def _outer(i):
        t0 = 2 * i
        _issue(t0 + 1, ridx1, rows1, sem1)
        _wait(ridx0, rows0, sem0)
        _acc(t0, rows0)
        _issue(t0 + 2, ridx0, rows0, sem0)
        _wait(ridx1, rows1, sem1)
        _acc(t0 + 1, rows1)

    _wait(ridx0, rows0, sem0)

    pltpu.sync_copy(win, osum_h.at[pl.ds(base, W)])
    pltpu.sync_copy(dwin, den_h.at[wid])


def _sk2b(xl, pc, ac, cnt):
    fn = pl.kernel(
        _sk2b_body,
        out_type=(jax.ShapeDtypeStruct((NT, H), jnp.float32),
                  jax.ShapeDtypeStruct((32, 3, 128), jnp.float32)),
        mesh=plsc.VectorSubcoreMesh(core_axis_name="c", subcore_axis_name="s"),
        scratch_types=(
            pltpu.VMEM((CAP,), jnp.int32),
            pltpu.VMEM((CAP,), jnp.float32),
            pltpu.VMEM((KD,), jnp.int32),
            pltpu.VMEM((KD,), jnp.int32),
            pltpu.VMEM((KD, H), jnp.float32),
            pltpu.VMEM((KD, H), jnp.float32),
            pltpu.VMEM((W, H), jnp.float32),
            pltpu.VMEM((3, 128), jnp.float32),
            pltpu.VMEM((16,), jnp.int32),
            pltpu.SemaphoreType.DMA,
            pltpu.SemaphoreType.DMA,
        ),
        compiler_params=pltpu.CompilerParams(needs_layout_passes=False),
    )
    return fn(xl, pc, ac, cnt)


NB = 16
NPAD = 10240
BLK = NPAD // NB


def _pool_ffn_body(h_ref, batch_ref, w1_ref, b1_ref, w2_ref, b2_ref,
                   out_ref, acc_ref, cnt_ref):
    i = pl.program_id(0)

    @pl.when(i == 0)
    def _init():
        acc_ref[...] = jnp.zeros_like(acc_ref)
        cnt_ref[...] = jnp.zeros_like(cnt_ref)

    hb = h_ref[...]
    bb = batch_ref[0, 0, :]
    onehot_t = (bb[None, :] == jax.lax.broadcasted_iota(jnp.int32, (B, BLK), 0)
                ).astype(jnp.float32)
    acc_ref[...] += jnp.dot(onehot_t, hb, preferred_element_type=jnp.float32)
    cnt_ref[...] += jnp.broadcast_to(
        jnp.sum(onehot_t, axis=1, keepdims=True), (B, H))

    @pl.when(i == NB - 1)
    def _fin():
        g = acc_ref[...] / jnp.maximum(cnt_ref[...], 1.0)
        g = jax.nn.relu(jnp.dot(g, w1_ref[...],
                                preferred_element_type=jnp.float32)
                        + b1_ref[...])
        out_ref[...] = jnp.dot(g, w2_ref[...],
                               preferred_element_type=jnp.float32) + b2_ref[...]


def _pool_ffn(h, batch, W1, b1, W2, b2):
    hp = jnp.zeros((NPAD, H), jnp.float32).at[:N].set(h)
    bp = jnp.full((NPAD,), -1, jnp.int32).at[:N].set(batch.astype(jnp.int32))
    bp = bp.reshape(NB, 1, BLK)
    w2p = jnp.zeros((H, 128), jnp.float32).at[:, :1].set(W2)
    b2p = jnp.zeros((1, 128), jnp.float32).at[:, :1].set(b2[None, :])
    out = pl.pallas_call(
        _pool_ffn_body,
        grid=(NB,),
        in_specs=[
            pl.BlockSpec((BLK, H), lambda i: (i, 0)),
            pl.BlockSpec((1, 1, BLK), lambda i: (i, 0, 0)),
            pl.BlockSpec((H, H), lambda i: (0, 0)),
            pl.BlockSpec((1, H), lambda i: (0, 0)),
            pl.BlockSpec((H, 128), lambda i: (0, 0)),
            pl.BlockSpec((1, 128), lambda i: (0, 0)),
        ],
        out_specs=pl.BlockSpec((B, 128), lambda i: (0, 0)),
        out_shape=jax.ShapeDtypeStruct((B, 128), jnp.float32),
        scratch_shapes=[pltpu.VMEM((B, H), jnp.float32),
                        pltpu.VMEM((B, H), jnp.float32)],
    )(hp, bp, W1, b1[None, :], w2p, b2p)
    return out[:, :1]


def kernel(x, edge_index, edge_attr, batch, Wn, bn, We0, be0, Wl, bl, Wr, br,
           Wea, att, bias, W1, b1, W2, b2):
    ei = edge_index.astype(jnp.int32)
    loop = jnp.arange(N, dtype=jnp.int32)
    row = jnp.concatenate([ei[0], loop,
                           jnp.zeros((EPP - EP,), jnp.int32)])
    col = jnp.concatenate([ei[1], loop,
                           jnp.full((EPP - EP,), N, jnp.int32)])

    h, xl, xr = _tk0(x, Wn, bn, Wl[0], bl[0], Wr[0], br[0])
    ea_mean = _tkmean(edge_attr, We0, be0)

    for l in range(DEPTH):
        eaw = _tkeaw(edge_attr, We0, be0, Wea[l], ea_mean)
        logit, mpart = _sk1(row, col, xl, xr, eaw, att[l])
        m = _tkmred(mpart)
        pc, ac, cnt = _sk2a(row, col, logit, m)
        osum, den = _sk2b(xl, pc, ac, cnt)
        den = den.reshape(32, 384)[:, :W].reshape(NT, 1)
        last = l == DEPTH - 1
        if last:
            h = _tkfin(osum, den, bias[l], Wl[0], bl[0], Wr[0], br[0], True)
        else:
            h, xl, xr = _tkfin(osum, den, bias[l], Wl[l + 1], bl[l + 1],
                               Wr[l + 1], br[l + 1], False)

    return _pool_ffn(h, batch, W1, b1, W2, b2)


# speedup vs baseline: 1.6177x; 1.0032x over previous
import functools

import jax
import jax.numpy as jnp
from jax import lax
from jax.experimental import pallas as pl
from jax.experimental.pallas import tpu as pltpu
from jax.experimental.pallas import tpu_sc as plsc

N = 10000
E = 320000
F_IN = 128
F_E = 16
H = 256
B = 64
DEPTH = 4

EP = E + N
EPP = 331776
C1 = EPP // 32
K = 64
NT = 10240
W = 320
CAP = 11520
DW = 16
SCK = 2048
KD = 32
NEG = -3.0e38

NBLK = 1000
EBLK = 512
NEBLK = E // EBLK
NEBLKP = EPP // EBLK


def _tk0_body(x_ref, wn_ref, bn_ref, wl_ref, bl_ref, wr_ref, br_ref,
              h_ref, xl_ref, xr_ref):
    h = jax.nn.relu(jnp.dot(x_ref[...], wn_ref[...],
                            preferred_element_type=jnp.float32) + bn_ref[...])
    h_ref[...] = h
    xl_ref[...] = jnp.dot(h, wl_ref[...],
                          preferred_element_type=jnp.float32) + bl_ref[...]
    xr_ref[...] = jnp.dot(h, wr_ref[...],
                          preferred_element_type=jnp.float32) + br_ref[...]


def _tk0(x, Wn, bn, Wl0, bl0, Wr0, br0):
    g = N // NBLK
    return pl.pallas_call(
        _tk0_body,
        grid=(g,),
        in_specs=[
            pl.BlockSpec((NBLK, F_IN), lambda i: (i, 0)),
            pl.BlockSpec((F_IN, H), lambda i: (0, 0)),
            pl.BlockSpec((1, H), lambda i: (0, 0)),
            pl.BlockSpec((H, H), lambda i: (0, 0)),
            pl.BlockSpec((1, H), lambda i: (0, 0)),
            pl.BlockSpec((H, H), lambda i: (0, 0)),
            pl.BlockSpec((1, H), lambda i: (0, 0)),
        ],
        out_specs=[pl.BlockSpec((NBLK, H), lambda i: (i, 0))] * 3,
        out_shape=[jax.ShapeDtypeStruct((N, H), jnp.float32)] * 3,
    )(x, Wn, bn[None, :], Wl0, bl0[None, :], Wr0, br0[None, :])


def _tkmean_body(ea_ref, we0_ref, be0_ref, out_ref, acc_ref):
    i = pl.program_id(0)

    @pl.when(i == 0)
    def _init():
        acc_ref[...] = jnp.zeros_like(acc_ref)

    ea0 = jax.nn.relu(jnp.dot(ea_ref[...], we0_ref[...],
                              preferred_element_type=jnp.float32) + be0_ref[...])
    acc_ref[...] += jnp.sum(ea0, axis=0, keepdims=True)

    @pl.when(i == NEBLK - 1)
    def _fin():
        out_ref[...] = acc_ref[...] * (1.0 / E)


def _tkmean(edge_attr, We0, be0):
    return pl.pallas_call(
        _tkmean_body,
        grid=(NEBLK,),
        in_specs=[
            pl.BlockSpec((EBLK, F_E), lambda i: (i, 0)),
            pl.BlockSpec((F_E, H), lambda i: (0, 0)),
            pl.BlockSpec((1, H), lambda i: (0, 0)),
        ],
        out_specs=pl.BlockSpec((1, H), lambda i: (0, 0)),
        out_shape=jax.ShapeDtypeStruct((1, H), jnp.float32),
        scratch_shapes=[pltpu.VMEM((1, H), jnp.float32)],
    )(edge_attr, We0, be0[None, :])


def _tkeaw_body(ea_ref, we0_ref, be0_ref, wea_ref, mean_ref, out_ref):
    i = pl.program_id(0)

    @pl.when(i < NEBLK)
    def _real():
        ea0 = jax.nn.relu(jnp.dot(ea_ref[...], we0_ref[...],
                                  preferred_element_type=jnp.float32)
                          + be0_ref[...])
        out_ref[...] = jnp.dot(ea0, wea_ref[...],
                               preferred_element_type=jnp.float32)

    @pl.when(i >= NEBLK)
    def _loops():
        mw = jnp.dot(mean_ref[...], wea_ref[...],
                     preferred_element_type=jnp.float32)
        out_ref[...] = jnp.broadcast_to(mw, (EBLK, H))


def _tkeaw(edge_attr, We0, be0, Wea_l, ea_mean):
    return pl.pallas_call(
        _tkeaw_body,
        grid=(NEBLKP,),
        in_specs=[
            pl.BlockSpec((EBLK, F_E), lambda i: (jnp.minimum(i, NEBLK - 1), 0)),
            pl.BlockSpec((F_E, H), lambda i: (0, 0)),
            pl.BlockSpec((1, H), lambda i: (0, 0)),
            pl.BlockSpec((H, H), lambda i: (0, 0)),
            pl.BlockSpec((1, H), lambda i: (0, 0)),
        ],
        out_specs=pl.BlockSpec((EBLK, H), lambda i: (i, 0)),
        out_shape=jax.ShapeDtypeStruct((EPP, H), jnp.float32),
    )(edge_attr, We0, be0[None, :], Wea_l, ea_mean)


def _tkmred_body(mpart_ref, m_ref):
    m_ref[...] = jnp.max(mpart_ref[...], axis=0, keepdims=True)


def _tkmred(mpart):
    return pl.pallas_call(
        _tkmred_body,
        grid=(1,),
        in_specs=[pl.BlockSpec((32, NT), lambda i: (0, 0))],
        out_specs=pl.BlockSpec((1, NT), lambda i: (0, 0)),
        out_shape=jax.ShapeDtypeStruct((1, NT), jnp.float32),
    )(mpart)


def _tkfin_body(osum_ref, den_ref, bias_ref, wl_ref, bl_ref, wr_ref, br_ref,
                h_ref, xl_ref=None, xr_ref=None, *, last):
    o = osum_ref[...]
    d = den_ref[...]
    xh = o / jnp.maximum(d, 1e-16) + bias_ref[...]
    h = (xh if last else jax.nn.relu(xh)) + xh
    h_ref[...] = h
    if not last:
        xl_ref[...] = jnp.dot(h, wl_ref[...],
                              preferred_element_type=jnp.float32) + bl_ref[...]
        xr_ref[...] = jnp.dot(h, wr_ref[...],
                              preferred_element_type=jnp.float32) + br_ref[...]


def _tkfin(osum, den, bias_l, Wl_n, bl_n, Wr_n, br_n, last):
    g = N // NBLK
    n_out = 1 if last else 3
    out_specs = [pl.BlockSpec((NBLK, H), lambda i: (i, 0))] * n_out
    out_shape = [jax.ShapeDtypeStruct((N, H), jnp.float32)] * n_out
    out = pl.pallas_call(
        functools.partial(_tkfin_body, last=last),
        grid=(g,),
        in_specs=[
            pl.BlockSpec((NBLK, H), lambda i: (i, 0)),
            pl.BlockSpec((NBLK, 1), lambda i: (i, 0)),
            pl.BlockSpec((1, H), lambda i: (0, 0)),
            pl.BlockSpec((H, H), lambda i: (0, 0)),
            pl.BlockSpec((1, H), lambda i: (0, 0)),
            pl.BlockSpec((H, H), lambda i: (0, 0)),
            pl.BlockSpec((1, H), lambda i: (0, 0)),
        ],
        out_specs=out_specs,
        out_shape=out_shape,
    )(osum, den, bias_l[None, :], Wl_n, bl_n[None, :], Wr_n, br_n[None, :])
    if last:
        return out[0] if isinstance(out, (list, tuple)) else out
    return out


def _gather16(vals, idx):
    return lax.gather(
        vals, idx[:, None],
        lax.GatherDimensionNumbers(offset_dims=(), collapsed_slice_dims=(0,),
                                   start_index_map=(0,)),
        (1,), mode=lax.GatherScatterMode.PROMISE_IN_BOUNDS)


def _iota16():
    return lax.broadcasted_iota(jnp.int32, (16,), 0)


def _lane_sum_all(v):
    for d in (1, 2, 4, 8):
        v = v + _gather16(v, jnp.bitwise_xor(_iota16(), d))
    return v


def _seg_scatter_max(table, col16, val16):
    iota = _iota16()
    fill = jnp.float32(NEG)
    acc = val16
    any_earlier = iota < 0
    for j in range(1, 16):
        idx = jnp.bitwise_and(iota - j, 15)
        rot_c = _gather16(col16, idx)
        rot_v = _gather16(val16, idx)
        same = rot_c == col16
        acc = jnp.maximum(acc, jnp.where(same, rot_v, fill))
        any_earlier = any_earlier | (same & (iota >= j))
    first = jnp.logical_not(any_earlier)
    cur = plsc.load_gather(table, [col16])
    plsc.store_scatter(table, [col16], jnp.maximum(cur, acc), mask=first)


def _sk1_body(row_h, col_h, xl_h, xr_h, eaw_h, att_h,
              logit_h, mpart_h,
              ridx0, cidx0, ridx1, cidx1, xlg0, xrg0, eaw0, xlg1, xrg1, eaw1,
              attv, lst0, lst1, mv, sem0, sem1):
    c = lax.axis_index("c")
    s = lax.axis_index("s")
    wid = s * 2 + c
    e0 = wid * C1
    pltpu.sync_copy(att_h, attv)

    @pl.loop(0, NT // 16)
    def _initm(i):
        mv[pl.ds(i * 16, 16)] = jnp.full((16,), NEG, jnp.float32)

    nchunks = C1 // K

    def _issue(sc, ridx, cidx, xlg, xrg, eawv, sem):
        base = e0 + jnp.minimum(sc, nchunks - 1) * K
        pltpu.sync_copy(row_h.at[pl.ds(base, K)], ridx)
        pltpu.sync_copy(col_h.at[pl.ds(base, K)], cidx)
        pltpu.async_copy(xl_h.at[ridx], xlg, sem)
        pltpu.async_copy(xr_h.at[cidx], xrg, sem)
        pltpu.async_copy(eaw_h.at[pl.ds(base, K)], eawv, sem)

    def _wait(ridx, cidx, xlg, xrg, eawv, sem):
        pltpu.make_async_copy(xl_h.at[ridx], xlg, sem).wait()
        pltpu.make_async_copy(xr_h.at[cidx], xrg, sem).wait()
        pltpu.make_async_copy(eaw_h.at[pl.ds(0, K)], eawv, sem).wait()

    def _compute(sc, cidx, xlg, xrg, eawv, lstage):
        base = e0 + sc * K

        @pl.loop(0, K // 16)
        def _q(q):
            lane = _iota16()
            logit16 = jnp.zeros((16,), jnp.float32)
            for j in range(16):
                jj = q * 16 + j
                acc = jnp.zeros((16,), jnp.float32)
                for v in range(16):
                    sl = pl.ds(v * 16, 16)
                    u = xlg[jj, sl] + xrg[jj, sl] + eawv[jj, sl]
                    u = jnp.maximum(u, 0.2 * u)
                    acc = acc + u * attv[sl]
                tot = _lane_sum_all(acc)
                logit16 = jnp.where(lane == j, tot, logit16)
            slq = pl.ds(q * 16, 16)
            lstage[slq] = logit16
            col16 = cidx[slq]
            _seg_scatter_max(mv, col16, logit16)

        pltpu.sync_copy(lstage, logit_h.at[pl.ds(base, K)])

    b0 = (ridx0, cidx0, xlg0, xrg0, eaw0, sem0)
    b1 = (ridx1, cidx1, xlg1, xrg1, eaw1, sem1)
    _issue(0, *b0[:-1], sem0)

    @pl.loop(0, nchunks // 2)
    def _outer(i):
        sc = 2 * i
        _issue(sc + 1, *b1[:-1], sem1)
        _wait(*b0[:-1], sem0)
        _compute(sc, cidx0, xlg0, xrg0, eaw0, lst0)
        _issue(sc + 2, *b0[:-1], sem0)
        _wait(*b1[:-1], sem1)
        _compute(sc + 1, cidx1, xlg1, xrg1, eaw1, lst1)

    _wait(*b0[:-1], sem0)
    pltpu.sync_copy(mv, mpart_h.at[wid])


def _sk1(row, col, xl, xr, eaw, att_l):
    fn = pl.kernel(
        _sk1_body,
        out_type=(jax.ShapeDtypeStruct((EPP,), jnp.float32),
                  jax.ShapeDtypeStruct((32, NT), jnp.float32)),
        mesh=plsc.VectorSubcoreMesh(core_axis_name="c", subcore_axis_name="s"),
        scratch_types=(
            pltpu.VMEM((K,), jnp.int32),
            pltpu.VMEM((K,), jnp.int32),
            pltpu.VMEM((K,), jnp.int32),
            pltpu.VMEM((K,), jnp.int32),
            pltpu.VMEM((K, H), jnp.float32),
            pltpu.VMEM((K, H), jnp.float32),
            pltpu.VMEM((K, H), jnp.float32),
            pltpu.VMEM((K, H), jnp.float32),
            pltpu.VMEM((K, H), jnp.float32),
            pltpu.VMEM((K, H), jnp.float32),
            pltpu.VMEM((H,), jnp.float32),
            pltpu.VMEM((K,), jnp.float32),
            pltpu.VMEM((K,), jnp.float32),
            pltpu.VMEM((NT,), jnp.float32),
            pltpu.SemaphoreType.DMA,
            pltpu.SemaphoreType.DMA,
        ),
        compiler_params=pltpu.CompilerParams(needs_layout_passes=False),
    )
    return fn(row, col, xl, xr, eaw, att_l)


def _sk2a_body(row_h, col_h, logit_h, m_h,
               pc_h, ac_h, cnt_h,
               rowb0, colb0, lb0, rowb1, colb1, lb1,
               packed, acomp, mv, stg, sem0, sem1):
    c = lax.axis_index("c")
    s = lax.axis_index("s")
    wid = s * 2 + c
    base = wid * W

    pltpu.sync_copy(m_h.at[0], mv)

    @pl.loop(0, CAP // 16)
    def _zc(i):
        sl = pl.ds(i * 16, 16)
        packed[sl] = jnp.zeros((16,), jnp.int32)
        acomp[sl] = jnp.zeros((16,), jnp.float32)

    nchunks = EPP // SCK

    def _issue(ci, bufs, sem):
        b0 = ci * SCK
        pltpu.async_copy(row_h.at[pl.ds(b0, SCK)], bufs[0], sem)
        pltpu.async_copy(col_h.at[pl.ds(b0, SCK)], bufs[1], sem)
        pltpu.async_copy(logit_h.at[pl.ds(b0, SCK)], bufs[2], sem)

    def _wait(ci, bufs, sem):
        b0 = ci * SCK
        pltpu.make_async_copy(row_h.at[pl.ds(b0, SCK)], bufs[0], sem).wait()
        pltpu.make_async_copy(col_h.at[pl.ds(b0, SCK)], bufs[1], sem).wait()
        pltpu.make_async_copy(logit_h.at[pl.ds(b0, SCK)], bufs[2], sem).wait()

    def _scan(bufs, off):
        rowb, colb, lb = bufs
        for q in range(SCK // 16):
            sl = pl.ds(q * 16, 16)
            col16 = colb[sl]
            row16 = rowb[sl]
            l16 = lb[sl]
            m16 = plsc.load_gather(mv, [col16])
            a16 = jnp.exp(l16 - m16)
            mask = (col16 >= base) & (col16 < base + W)
            packed16 = row16 * 512 + (col16 - base)
            plsc.store_compressed(packed.at[pl.ds(off, 16)], packed16,
                                  mask=mask)
            plsc.store_compressed(acomp.at[pl.ds(off, 16)], a16, mask=mask)
            cnt = plsc.all_reduce_population_count(mask)[0]
            off = jnp.minimum(off + cnt, CAP - 2 * KD)
        return off

    bufs0 = (rowb0, colb0, lb0)
    bufs1 = (rowb1, colb1, lb1)
    _issue(0, bufs0, sem0)

    def _outer(i, off):
        c0 = 2 * i
        _issue(jnp.minimum(c0 + 1, nchunks - 1), bufs1, sem1)
        _wait(c0, bufs0, sem0)
        off = _scan(bufs0, off)
        _issue(jnp.minimum(c0 + 2, nchunks - 1), bufs0, sem0)
        _wait(jnp.minimum(c0 + 1, nchunks - 1), bufs1, sem1)
        off = _scan(bufs1, off)
        return off

    nh = pl.loop(0, nchunks // 2, init_carry=jnp.int32(0))(_outer)
    _wait(nchunks - 1, bufs0, sem0)

    lane = _iota16()
    stg[pl.ds(0, 16)] = jnp.where(lane == 0, nh, 0)
    pltpu.sync_copy(stg, cnt_h.at[wid])
    pltpu.sync_copy(packed, pc_h.at[wid])
    pltpu.sync_copy(acomp, ac_h.at[wid])


def _sk2a(row, col, logit, m):
    fn = pl.kernel(
        _sk2a_body,
        out_type=(jax.ShapeDtypeStruct((32, CAP), jnp.int32),
                  jax.ShapeDtypeStruct((32, CAP), jnp.float32),
                  jax.ShapeDtypeStruct((32, 16), jnp.int32)),
        mesh=plsc.VectorSubcoreMesh(core_axis_name="c", subcore_axis_name="s"),
        scratch_types=(
            pltpu.VMEM((SCK,), jnp.int32),
            pltpu.VMEM((SCK,), jnp.int32),
            pltpu.VMEM((SCK,), jnp.float32),
            pltpu.VMEM((SCK,), jnp.int32),
            pltpu.VMEM((SCK,), jnp.int32),
            pltpu.VMEM((SCK,), jnp.float32),
            pltpu.VMEM((CAP,), jnp.int32),
            pltpu.VMEM((CAP,), jnp.float32),
            pltpu.VMEM((NT,), jnp.float32),
            pltpu.VMEM((16,), jnp.int32),
            pltpu.SemaphoreType.DMA,
            pltpu.SemaphoreType.DMA,
        ),
        compiler_params=pltpu.CompilerParams(needs_layout_passes=False),
    )
    return fn(row, col, logit, m)


def _sk2b_body(xl_h, pc_h, ac_h, cnt_h,
               osum_h, den_h,
               packed, acomp, ridx0, ridx1, rows0, rows1, win, dwin, stg,
               sem0, sem1):
    c = lax.axis_index("c")
    s = lax.axis_index("s")
    wid = s * 2 + c
    base = wid * W

    pltpu.sync_copy(cnt_h.at[wid], stg)
    nh = stg[pl.ds(0, 16)][0]
    pltpu.sync_copy(pc_h.at[wid], packed)
    pltpu.sync_copy(ac_h.at[wid], acomp)

    @pl.loop(0, W)
    def _zw(r):
        for v in range(H // 16):
            win[r, pl.ds(v * 16, 16)] = jnp.zeros((16,), jnp.float32)

    @pl.loop(0, 3)
    def _zd(r):
        for v in range(8):
            dwin[r, pl.ds(v * 16, 16)] = jnp.zeros((16,), jnp.float32)

    ndr = (nh + KD - 1) // KD
    maxj0 = CAP - KD

    def _issue(t, ridx, rows, sem):
        j0 = jnp.minimum(t * KD, maxj0)
        for q in range(KD // 16):
            sl = pl.ds(j0 + q * 16, 16)
            ridx[pl.ds(q * 16, 16)] = jnp.right_shift(packed[sl], 9)
        pltpu.async_copy(xl_h.at[ridx], rows, sem)

    def _wait(ridx, rows, sem):
        pltpu.make_async_copy(xl_h.at[ridx], rows, sem).wait()

    def _acc(t, rows):
        j0 = jnp.minimum(t * KD, maxj0)
        lane = _iota16()
        for q in range(KD // 16):
            sl = pl.ds(j0 + q * 16, 16)
            cl16 = jnp.bitwise_and(packed[sl], 511)
            a16 = acomp[sl]
            for j in range(16):
                aj = a16[j]
                cj = cl16[j]
                jj = q * 16 + j
                for v in range(H // 16):
                    slv = pl.ds(v * 16, 16)
                    win[cj, slv] = win[cj, slv] + rows[jj, slv] * aj
                cjr = jnp.right_shift(cj, 7)
                cjc = jnp.bitwise_and(cj, 127)
                cjc0 = cjc - jnp.bitwise_and(cjc, 15)
                lt = jnp.bitwise_and(cjc, 15)
                sld = pl.ds(cjc0, 16)
                dwin[cjr, sld] = dwin[cjr, sld] + jnp.where(lane == lt, aj,
                                                            0.0)

    nouter = (ndr + 1) // 2
    _issue(0, ridx0, rows0, sem0)

    @pl.loop(0, nouter)
    def _outer(i):
        t0 = 2 * i
        _issue(t0 + 1, ridx1, rows1, sem1)
        _wait(ridx0, rows0, sem0)
        _acc(t0, rows0)
        _issue(t0 + 2, ridx0, rows0, sem0)
        _wait(ridx1, rows1, sem1)
        _acc(t0 + 1, rows1)

    _wait(ridx0, rows0, sem0)

    pltpu.sync_copy(win, osum_h.at[pl.ds(base, W)])
    pltpu.sync_copy(dwin, den_h.at[wid])


def _sk2b(xl, pc, ac, cnt):
    fn = pl.kernel(
        _sk2b_body,
        out_type=(jax.ShapeDtypeStruct((NT, H), jnp.float32),
                  jax.ShapeDtypeStruct((32, 3, 128), jnp.float32)),
        mesh=plsc.VectorSubcoreMesh(core_axis_name="c", subcore_axis_name="s"),
        scratch_types=(
            pltpu.VMEM((CAP,), jnp.int32),
            pltpu.VMEM((CAP,), jnp.float32),
            pltpu.VMEM((KD,), jnp.int32),
            pltpu.VMEM((KD,), jnp.int32),
            pltpu.VMEM((KD, H), jnp.float32),
            pltpu.VMEM((KD, H), jnp.float32),
            pltpu.VMEM((W, H), jnp.float32),
            pltpu.VMEM((3, 128), jnp.float32),
            pltpu.VMEM((16,), jnp.int32),
            pltpu.SemaphoreType.DMA,
            pltpu.SemaphoreType.DMA,
        ),
        compiler_params=pltpu.CompilerParams(needs_layout_passes=False),
    )
    return fn(xl, pc, ac, cnt)


NB = 16
NPAD = 10240
BLK = NPAD // NB


def _pool_ffn_body(h_ref, batch_ref, w1_ref, b1_ref, w2_ref, b2_ref,
                   out_ref, acc_ref, cnt_ref):
    i = pl.program_id(0)

    @pl.when(i == 0)
    def _init():
        acc_ref[...] = jnp.zeros_like(acc_ref)
        cnt_ref[...] = jnp.zeros_like(cnt_ref)

    hb = h_ref[...]
    bb = batch_ref[0, 0, :]
    onehot_t = (bb[None, :] == jax.lax.broadcasted_iota(jnp.int32, (B, BLK), 0)
                ).astype(jnp.float32)
    acc_ref[...] += jnp.dot(onehot_t, hb, preferred_element_type=jnp.float32)
    cnt_ref[...] += jnp.broadcast_to(
        jnp.sum(onehot_t, axis=1, keepdims=True), (B, H))

    @pl.when(i == NB - 1)
    def _fin():
        g = acc_ref[...] / jnp.maximum(cnt_ref[...], 1.0)
        g = jax.nn.relu(jnp.dot(g, w1_ref[...],
                                preferred_element_type=jnp.float32)
                        + b1_ref[...])
        out_ref[...] = jnp.dot(g, w2_ref[...],
                               preferred_element_type=jnp.float32) + b2_ref[...]


def _pool_ffn(h, batch, W1, b1, W2, b2):
    hp = jnp.zeros((NPAD, H), jnp.float32).at[:N].set(h)
    bp = jnp.full((NPAD,), -1, jnp.int32).at[:N].set(batch.astype(jnp.int32))
    bp = bp.reshape(NB, 1, BLK)
    w2p = jnp.zeros((H, 128), jnp.float32).at[:, :1].set(W2)
    b2p = jnp.zeros((1, 128), jnp.float32).at[:, :1].set(b2[None, :])
    out = pl.pallas_call(
        _pool_ffn_body,
        grid=(NB,),
        in_specs=[
            pl.BlockSpec((BLK, H), lambda i: (i, 0)),
            pl.BlockSpec((1, 1, BLK), lambda i: (i, 0, 0)),
            pl.BlockSpec((H, H), lambda i: (0, 0)),
            pl.BlockSpec((1, H), lambda i: (0, 0)),
            pl.BlockSpec((H, 128), lambda i: (0, 0)),
            pl.BlockSpec((1, 128), lambda i: (0, 0)),
        ],
        out_specs=pl.BlockSpec((B, 128), lambda i: (0, 0)),
        out_shape=jax.ShapeDtypeStruct((B, 128), jnp.float32),
        scratch_shapes=[pltpu.VMEM((B, H), jnp.float32),
                        pltpu.VMEM((B, H), jnp.float32)],
    )(hp, bp, W1, b1[None, :], w2p, b2p)
    return out[:, :1]


def kernel(x, edge_index, edge_attr, batch, Wn, bn, We0, be0, Wl, bl, Wr, br,
           Wea, att, bias, W1, b1, W2, b2):
    ei = edge_index.astype(jnp.int32)
    loop = jnp.arange(N, dtype=jnp.int32)
    row = jnp.concatenate([ei[0], loop,
                           jnp.zeros((EPP - EP,), jnp.int32)])
    col = jnp.concatenate([ei[1], loop,
                           jnp.full((EPP - EP,), N, jnp.int32)])

    h, xl, xr = _tk0(x, Wn, bn, Wl[0], bl[0], Wr[0], br[0])
    ea_mean = _tkmean(edge_attr, We0, be0)

    for l in range(DEPTH):
        eaw = _tkeaw(edge_attr, We0, be0, Wea[l], ea_mean)
        logit, mpart = _sk1(row, col, xl, xr, eaw, att[l])
        m = _tkmred(mpart)
        pc, ac, cnt = _sk2a(row, col, logit, m)
        osum, den = _sk2b(xl, pc, ac, cnt)
        den = den.reshape(32, 384)[:, :W].reshape(NT, 1)
        last = l == DEPTH - 1
        if last:
            h = _tkfin(osum, den, bias[l], Wl[0], bl[0], Wr[0], br[0], True)
        else:
            h, xl, xr = _tkfin(osum, den, bias[l], Wl[l + 1], bl[l + 1],
                               Wr[l + 1], br[l + 1], False)

    return _pool_ffn(h, batch, W1, b1, W2, b2)


# speedup vs baseline: 1.9983x; 1.2353x over previous
import functools

import jax
import jax.numpy as jnp
from jax import lax
from jax.experimental import pallas as pl
from jax.experimental.pallas import tpu as pltpu
from jax.experimental.pallas import tpu_sc as plsc

N = 10000
E = 320000
F_IN = 128
F_E = 16
H = 256
B = 64
DEPTH = 4

EP = E + N
EPP = 331776
C1 = EPP // 32
K = 64
NT = 10240
W = 320
CAP = 11520
DW = 16
SCK = 2048
KD = 32
NEG = -3.0e38

NBLK = 1000
EBLK = 512
NEBLK = E // EBLK
NEBLKP = EPP // EBLK


def _tk0_body(x_ref, wn_ref, bn_ref, wl_ref, bl_ref, wr_ref, br_ref,
              h_ref, xl_ref, xr_ref):
    h = jax.nn.relu(jnp.dot(x_ref[...], wn_ref[...],
                            preferred_element_type=jnp.float32) + bn_ref[...])
    h_ref[...] = h
    xl_ref[...] = jnp.dot(h, wl_ref[...],
                          preferred_element_type=jnp.float32) + bl_ref[...]
    xr_ref[...] = jnp.dot(h, wr_ref[...],
                          preferred_element_type=jnp.float32) + br_ref[...]


def _tk0(x, Wn, bn, Wl0, bl0, Wr0, br0):
    g = N // NBLK
    return pl.pallas_call(
        _tk0_body,
        grid=(g,),
        in_specs=[
            pl.BlockSpec((NBLK, F_IN), lambda i: (i, 0)),
            pl.BlockSpec((F_IN, H), lambda i: (0, 0)),
            pl.BlockSpec((1, H), lambda i: (0, 0)),
            pl.BlockSpec((H, H), lambda i: (0, 0)),
            pl.BlockSpec((1, H), lambda i: (0, 0)),
            pl.BlockSpec((H, H), lambda i: (0, 0)),
            pl.BlockSpec((1, H), lambda i: (0, 0)),
        ],
        out_specs=[pl.BlockSpec((NBLK, H), lambda i: (i, 0))] * 3,
        out_shape=[jax.ShapeDtypeStruct((N, H), jnp.float32)] * 3,
    )(x, Wn, bn[None, :], Wl0, bl0[None, :], Wr0, br0[None, :])


def _tkmean_body(ea_ref, we0_ref, be0_ref, out_ref, acc_ref):
    i = pl.program_id(0)

    @pl.when(i == 0)
    def _init():
        acc_ref[...] = jnp.zeros_like(acc_ref)

    ea0 = jax.nn.relu(jnp.dot(ea_ref[...], we0_ref[...],
                              preferred_element_type=jnp.float32) + be0_ref[...])
    acc_ref[...] += jnp.sum(ea0, axis=0, keepdims=True)

    @pl.when(i == NEBLK - 1)
    def _fin():
        out_ref[...] = acc_ref[...] * (1.0 / E)


def _tkmean(edge_attr, We0, be0):
    return pl.pallas_call(
        _tkmean_body,
        grid=(NEBLK,),
        in_specs=[
            pl.BlockSpec((EBLK, F_E), lambda i: (i, 0)),
            pl.BlockSpec((F_E, H), lambda i: (0, 0)),
            pl.BlockSpec((1, H), lambda i: (0, 0)),
        ],
        out_specs=pl.BlockSpec((1, H), lambda i: (0, 0)),
        out_shape=jax.ShapeDtypeStruct((1, H), jnp.float32),
        scratch_shapes=[pltpu.VMEM((1, H), jnp.float32)],
    )(edge_attr, We0, be0[None, :])


def _tkeaw_body(ea_ref, we0_ref, be0_ref, wea_ref, mean_ref, out_ref):
    i = pl.program_id(0)

    @pl.when(i < NEBLK)
    def _real():
        ea0 = jax.nn.relu(jnp.dot(ea_ref[...], we0_ref[...],
                                  preferred_element_type=jnp.float32)
                          + be0_ref[...])
        out_ref[...] = jnp.dot(ea0, wea_ref[...],
                               preferred_element_type=jnp.float32)

    @pl.when(i >= NEBLK)
    def _loops():
        mw = jnp.dot(mean_ref[...], wea_ref[...],
                     preferred_element_type=jnp.float32)
        out_ref[...] = jnp.broadcast_to(mw, (EBLK, H))


def _tkeaw(edge_attr, We0, be0, Wea_l, ea_mean):
    return pl.pallas_call(
        _tkeaw_body,
        grid=(NEBLKP,),
        in_specs=[
            pl.BlockSpec((EBLK, F_E), lambda i: (jnp.minimum(i, NEBLK - 1), 0)),
            pl.BlockSpec((F_E, H), lambda i: (0, 0)),
            pl.BlockSpec((1, H), lambda i: (0, 0)),
            pl.BlockSpec((H, H), lambda i: (0, 0)),
            pl.BlockSpec((1, H), lambda i: (0, 0)),
        ],
        out_specs=pl.BlockSpec((EBLK, H), lambda i: (i, 0)),
        out_shape=jax.ShapeDtypeStruct((EPP, H), jnp.float32),
    )(edge_attr, We0, be0[None, :], Wea_l, ea_mean)


def _tkfin_body(osum_ref, den_ref, bias_ref, wl_ref, bl_ref, wr_ref, br_ref,
                h_ref, xl_ref=None, xr_ref=None, *, last):
    o = osum_ref[...]
    d = den_ref[...]
    xh = o / jnp.maximum(d, 1e-16) + bias_ref[...]
    h = (xh if last else jax.nn.relu(xh)) + xh
    h_ref[...] = h
    if not last:
        xl_ref[...] = jnp.dot(h, wl_ref[...],
                              preferred_element_type=jnp.float32) + bl_ref[...]
        xr_ref[...] = jnp.dot(h, wr_ref[...],
                              preferred_element_type=jnp.float32) + br_ref[...]


def _tkfin(osum, den, bias_l, Wl_n, bl_n, Wr_n, br_n, last):
    g = N // NBLK
    n_out = 1 if last else 3
    out_specs = [pl.BlockSpec((NBLK, H), lambda i: (i, 0))] * n_out
    out_shape = [jax.ShapeDtypeStruct((N, H), jnp.float32)] * n_out
    out = pl.pallas_call(
        functools.partial(_tkfin_body, last=last),
        grid=(g,),
        in_specs=[
            pl.BlockSpec((NBLK, H), lambda i: (i, 0)),
            pl.BlockSpec((NBLK, 1), lambda i: (i, 0)),
            pl.BlockSpec((1, H), lambda i: (0, 0)),
            pl.BlockSpec((H, H), lambda i: (0, 0)),
            pl.BlockSpec((1, H), lambda i: (0, 0)),
            pl.BlockSpec((H, H), lambda i: (0, 0)),
            pl.BlockSpec((1, H), lambda i: (0, 0)),
        ],
        out_specs=out_specs,
        out_shape=out_shape,
    )(osum, den, bias_l[None, :], Wl_n, bl_n[None, :], Wr_n, br_n[None, :])
    if last:
        return out[0] if isinstance(out, (list, tuple)) else out
    return out


def _gather16(vals, idx):
    return lax.gather(
        vals, idx[:, None],
        lax.GatherDimensionNumbers(offset_dims=(), collapsed_slice_dims=(0,),
                                   start_index_map=(0,)),
        (1,), mode=lax.GatherScatterMode.PROMISE_IN_BOUNDS)


def _iota16():
    return lax.broadcasted_iota(jnp.int32, (16,), 0)


def _lane_sum_all(v):
    for d in (1, 2, 4, 8):
        v = v + _gather16(v, jnp.bitwise_xor(_iota16(), d))
    return v


def _seg_scatter_max(table, col16, val16):
    iota = _iota16()
    fill = jnp.float32(NEG)
    acc = val16
    any_earlier = iota < 0
    for j in range(1, 16):
        idx = jnp.bitwise_and(iota - j, 15)
        rot_c = _gather16(col16, idx)
        rot_v = _gather16(val16, idx)
        same = rot_c == col16
        acc = jnp.maximum(acc, jnp.where(same, rot_v, fill))
        any_earlier = any_earlier | (same & (iota >= j))
    first = jnp.logical_not(any_earlier)
    cur = plsc.load_gather(table, [col16])
    plsc.store_scatter(table, [col16], jnp.maximum(cur, acc), mask=first)


def _scg_body(row_h, col_h, xl_h, xr_h,
              gxl_h, gxr_h,
              ridx0, cidx0, ridx1, cidx1, xlg0, xrg0, xlg1, xrg1,
              sem0, sem1):
    c = lax.axis_index("c")
    s = lax.axis_index("s")
    wid = s * 2 + c
    e0 = wid * C1
    nchunks = C1 // K

    def _issue(sc, ridx, cidx, xlg, xrg, sem):
        base = e0 + jnp.minimum(sc, nchunks - 1) * K
        pltpu.sync_copy(row_h.at[pl.ds(base, K)], ridx)
        pltpu.sync_copy(col_h.at[pl.ds(base, K)], cidx)
        pltpu.async_copy(xl_h.at[ridx], xlg, sem)
        pltpu.async_copy(xr_h.at[cidx], xrg, sem)

    def _wait(ridx, cidx, xlg, xrg, sem):
        pltpu.make_async_copy(xl_h.at[ridx], xlg, sem).wait()
        pltpu.make_async_copy(xr_h.at[cidx], xrg, sem).wait()

    def _flush(sc, xlg, xrg):
        base = e0 + sc * K
        pltpu.sync_copy(xlg, gxl_h.at[pl.ds(base, K)])
        pltpu.sync_copy(xrg, gxr_h.at[pl.ds(base, K)])

    _issue(0, ridx0, cidx0, xlg0, xrg0, sem0)

    @pl.loop(0, nchunks // 2)
    def _outer(i):
        sc = 2 * i
        _issue(sc + 1, ridx1, cidx1, xlg1, xrg1, sem1)
        _wait(ridx0, cidx0, xlg0, xrg0, sem0)
        _flush(sc, xlg0, xrg0)
        _issue(sc + 2, ridx0, cidx0, xlg0, xrg0, sem0)
        _wait(ridx1, cidx1, xlg1, xrg1, sem1)
        _flush(sc + 1, xlg1, xrg1)

    _wait(ridx0, cidx0, xlg0, xrg0, sem0)


def _scg(row, col, xl, xr):
    fn = pl.kernel(
        _scg_body,
        out_type=(jax.ShapeDtypeStruct((EPP, H), jnp.float32),
                  jax.ShapeDtypeStruct((EPP, H), jnp.float32)),
        mesh=plsc.VectorSubcoreMesh(core_axis_name="c", subcore_axis_name="s"),
        scratch_types=(
            pltpu.VMEM((K,), jnp.int32),
            pltpu.VMEM((K,), jnp.int32),
            pltpu.VMEM((K,), jnp.int32),
            pltpu.VMEM((K,), jnp.int32),
            pltpu.VMEM((K, H), jnp.float32),
            pltpu.VMEM((K, H), jnp.float32),
            pltpu.VMEM((K, H), jnp.float32),
            pltpu.VMEM((K, H), jnp.float32),
            pltpu.SemaphoreType.DMA,
            pltpu.SemaphoreType.DMA,
        ),
        compiler_params=pltpu.CompilerParams(needs_layout_passes=False),
    )
    return fn(row, col, xl, xr)


LBLK = 4096


def _tklogit_body(gxl_ref, gxr_ref, eaw_ref, att_ref, out_ref):
    u = gxl_ref[...] + gxr_ref[...] + eaw_ref[...]
    u = jnp.maximum(u, 0.2 * u)
    lg = jnp.sum(u * att_ref[...], axis=1)
    out_ref[...] = lg.reshape(LBLK // 512, 512)


def _tklogit(gxl, gxr, eaw, att_l):
    g = EPP // LBLK
    out = pl.pallas_call(
        _tklogit_body,
        grid=(g,),
        in_specs=[
            pl.BlockSpec((LBLK, H), lambda i: (i, 0)),
            pl.BlockSpec((LBLK, H), lambda i: (i, 0)),
            pl.BlockSpec((LBLK, H), lambda i: (i, 0)),
            pl.BlockSpec((1, H), lambda i: (0, 0)),
        ],
        out_specs=pl.BlockSpec((LBLK // 512, 512), lambda i: (i, 0)),
        out_shape=jax.ShapeDtypeStruct((EPP // 512, 512), jnp.float32),
    )(gxl, gxr, eaw, att_l[None, :])
    return out.reshape(EPP)


def _sk2a_body(row_h, col_h, logit_h,
               pc_h, ac_h, cnt_h,
               rowb0, colb0, lb0, rowb1, colb1, lb1,
               packed, acomp, mwin, stg, sem0, sem1):
    c = lax.axis_index("c")
    s = lax.axis_index("s")
    wid = s * 2 + c
    base = wid * W

    @pl.loop(0, 32)
    def _zm(i):
        mwin[pl.ds(i * 16, 16)] = jnp.full((16,), NEG, jnp.float32)

    @pl.loop(0, CAP // 16)
    def _zc(i):
        sl = pl.ds(i * 16, 16)
        packed[sl] = jnp.zeros((16,), jnp.int32)
        acomp[sl] = jnp.zeros((16,), jnp.float32)

    nchunks = EPP // SCK

    def _issue(ci, bufs, sem):
        b0 = ci * SCK
        pltpu.async_copy(row_h.at[pl.ds(b0, SCK)], bufs[0], sem)
        pltpu.async_copy(col_h.at[pl.ds(b0, SCK)], bufs[1], sem)
        pltpu.async_copy(logit_h.at[pl.ds(b0, SCK)], bufs[2], sem)

    def _wait(ci, bufs, sem):
        b0 = ci * SCK
        pltpu.make_async_copy(row_h.at[pl.ds(b0, SCK)], bufs[0], sem).wait()
        pltpu.make_async_copy(col_h.at[pl.ds(b0, SCK)], bufs[1], sem).wait()
        pltpu.make_async_copy(logit_h.at[pl.ds(b0, SCK)], bufs[2], sem).wait()

    def _scan(bufs, off):
        rowb, colb, lb = bufs

        def _vec(q, off):
            sl = pl.ds(q * 16, 16)
            col16 = colb[sl]
            row16 = rowb[sl]
            l16 = lb[sl]
            mask = (col16 >= base) & (col16 < base + W)
            cloc = jnp.where(mask, col16 - base, 511)
            _seg_scatter_max(mwin, cloc, jnp.where(mask, l16,
                                                   jnp.float32(NEG)))
            packed16 = row16 * 512 + cloc
            plsc.store_compressed(packed.at[pl.ds(off, 16)], packed16,
                                  mask=mask)
            plsc.store_compressed(acomp.at[pl.ds(off, 16)], l16, mask=mask)
            cnt = plsc.all_reduce_population_count(mask)[0]
            return jnp.minimum(off + cnt, CAP - 2 * KD)

        return pl.loop(0, SCK // 16, init_carry=off)(_vec)

    bufs0 = (rowb0, colb0, lb0)
    bufs1 = (rowb1, colb1, lb1)
    _issue(0, bufs0, sem0)

    def _outer(i, off):
        c0 = 2 * i
        _issue(jnp.minimum(c0 + 1, nchunks - 1), bufs1, sem1)
        _wait(c0, bufs0, sem0)
        off = _scan(bufs0, off)
        _issue(jnp.minimum(c0 + 2, nchunks - 1), bufs0, sem0)
        _wait(jnp.minimum(c0 + 1, nchunks - 1), bufs1, sem1)
        off = _scan(bufs1, off)
        return off

    nh = pl.loop(0, nchunks // 2, init_carry=jnp.int32(0))(_outer)
    _wait(nchunks - 1, bufs0, sem0)

    lane = _iota16()

    @pl.loop(0, (nh + 15) // 16)
    def _phb(t):
        sl = pl.ds(t * 16, 16)
        cloc16 = jnp.bitwise_and(packed[sl], 511)
        m16 = plsc.load_gather(mwin, [cloc16])
        a16 = jnp.exp(acomp[sl] - m16)
        gl = t * 16 + lane
        acomp[sl] = jnp.where(gl < nh, a16, 0.0)

    stg[pl.ds(0, 16)] = jnp.where(lane == 0, nh, 0)
    pltpu.sync_copy(stg, cnt_h.at[wid])
    pltpu.sync_copy(packed, pc_h.at[wid])
    pltpu.sync_copy(acomp, ac_h.at[wid])


def _sk2a(row, col, logit):
    fn = pl.kernel(
        _sk2a_body,
        out_type=(jax.ShapeDtypeStruct((32, CAP), jnp.int32),
                  jax.ShapeDtypeStruct((32, CAP), jnp.float32),
                  jax.ShapeDtypeStruct((32, 16), jnp.int32)),
        mesh=plsc.VectorSubcoreMesh(core_axis_name="c", subcore_axis_name="s"),
        scratch_types=(
            pltpu.VMEM((SCK,), jnp.int32),
            pltpu.VMEM((SCK,), jnp.int32),
            pltpu.VMEM((SCK,), jnp.float32),
            pltpu.VMEM((SCK,), jnp.int32),
            pltpu.VMEM((SCK,), jnp.int32),
            pltpu.VMEM((SCK,), jnp.float32),
            pltpu.VMEM((CAP,), jnp.int32),
            pltpu.VMEM((CAP,), jnp.float32),
            pltpu.VMEM((512,), jnp.float32),
            pltpu.VMEM((16,), jnp.int32),
            pltpu.SemaphoreType.DMA,
            pltpu.SemaphoreType.DMA,
        ),
        compiler_params=pltpu.CompilerParams(needs_layout_passes=False),
    )
    return fn(row, col, logit)


def _sk2b_body(xl_h, pc_h, ac_h, cnt_h,
               osum_h, den_h,
               packed, acomp, ridx0, ridx1, rows0, rows1, win, dwin, stg,
               sem0, sem1):
    c = lax.axis_index("c")
    s = lax.axis_index("s")
    wid = s * 2 + c
    base = wid * W

    pltpu.sync_copy(cnt_h.at[wid], stg)
    nh = stg[pl.ds(0, 16)][0]
    pltpu.sync_copy(pc_h.at[wid], packed)
    pltpu.sync_copy(ac_h.at[wid], acomp)

    @pl.loop(0, W)
    def _zw(r):
        for v in range(H // 16):
            win[r, pl.ds(v * 16, 16)] = jnp.zeros((16,), jnp.float32)

    @pl.loop(0, 3)
    def _zd(r):
        for v in range(8):
            dwin[r, pl.ds(v * 16, 16)] = jnp.zeros((16,), jnp.float32)

    ndr = (nh + KD - 1) // KD
    maxj0 = CAP - KD

    def _issue(t, ridx, rows, sem):
        j0 = jnp.minimum(t * KD, maxj0)
        for q in range(KD // 16):
            sl = pl.ds(j0 + q * 16, 16)
            ridx[pl.ds(q * 16, 16)] = jnp.right_shift(packed[sl], 9)
        pltpu.async_copy(xl_h.at[ridx], rows, sem)

    def _wait(ridx, rows, sem):
        pltpu.make_async_copy(xl_h.at[ridx], rows, sem).wait()

    def _acc(t, rows):
        j0 = jnp.minimum(t * KD, maxj0)
        lane = _iota16()
        for q in range(KD // 16):
            sl = pl.ds(j0 + q * 16, 16)
            cl16 = jnp.bitwise_and(packed[sl], 511)
            a16 = acomp[sl]
            for j in range(16):
                aj = a16[j]
                cj = cl16[j]
                jj = q * 16 + j
                for v in range(H // 16):
                    slv = pl.ds(v * 16, 16)
                    win[cj, slv] = win[cj, slv] + rows[jj, slv] * aj
                cjr = jnp.right_shift(cj, 7)
                cjc = jnp.bitwise_and(cj, 127)
                cjc0 = cjc - jnp.bitwise_and(cjc, 15)
                lt = jnp.bitwise_and(cjc, 15)
                sld = pl.ds(cjc0, 16)
                dwin[cjr, sld] = dwin[cjr, sld] + jnp.where(lane == lt, aj,
                                                            0.0)

    nouter = (ndr + 1) // 2
    _issue(0, ridx0, rows0, sem0)

    @pl.loop(0, nouter)
    def _outer(i):
        t0 = 2 * i
        _issue(t0 + 1, ridx1, rows1, sem1)
        _wait(ridx0, rows0, sem0)
        _acc(t0, rows0)
        _issue(t0 + 2, ridx0, rows0, sem0)
        _wait(ridx1, rows1, sem1)
        _acc(t0 + 1, rows1)

    _wait(ridx0, rows0, sem0)

    pltpu.sync_copy(win, osum_h.at[pl.ds(base, W)])
    pltpu.sync_copy(dwin, den_h.at[wid])


def _sk2b(xl, pc, ac, cnt):
    fn = pl.kernel(
        _sk2b_body,
        out_type=(jax.ShapeDtypeStruct((NT, H), jnp.float32),
                  jax.ShapeDtypeStruct((32, 3, 128), jnp.float32)),
        mesh=plsc.VectorSubcoreMesh(core_axis_name="c", subcore_axis_name="s"),
        scratch_types=(
            pltpu.VMEM((CAP,), jnp.int32),
            pltpu.VMEM((CAP,), jnp.float32),
            pltpu.VMEM((KD,), jnp.int32),
            pltpu.VMEM((KD,), jnp.int32),
            pltpu.VMEM((KD, H), jnp.float32),
            pltpu.VMEM((KD, H), jnp.float32),
            pltpu.VMEM((W, H), jnp.float32),
            pltpu.VMEM((3, 128), jnp.float32),
            pltpu.VMEM((16,), jnp.int32),
            pltpu.SemaphoreType.DMA,
            pltpu.SemaphoreType.DMA,
        ),
        compiler_params=pltpu.CompilerParams(needs_layout_passes=False),
    )
    return fn(xl, pc, ac, cnt)


NB = 16
NPAD = 10240
BLK = NPAD // NB


def _pool_ffn_body(h_ref, batch_ref, w1_ref, b1_ref, w2_ref, b2_ref,
                   out_ref, acc_ref, cnt_ref):
    i = pl.program_id(0)

    @pl.when(i == 0)
    def _init():
        acc_ref[...] = jnp.zeros_like(acc_ref)
        cnt_ref[...] = jnp.zeros_like(cnt_ref)

    hb = h_ref[...]
    bb = batch_ref[0, 0, :]
    onehot_t = (bb[None, :] == jax.lax.broadcasted_iota(jnp.int32, (B, BLK), 0)
                ).astype(jnp.float32)
    acc_ref[...] += jnp.dot(onehot_t, hb, preferred_element_type=jnp.float32)
    cnt_ref[...] += jnp.broadcast_to(
        jnp.sum(onehot_t, axis=1, keepdims=True), (B, H))

    @pl.when(i == NB - 1)
    def _fin():
        g = acc_ref[...] / jnp.maximum(cnt_ref[...], 1.0)
        g = jax.nn.relu(jnp.dot(g, w1_ref[...],
                                preferred_element_type=jnp.float32)
                        + b1_ref[...])
        out_ref[...] = jnp.dot(g, w2_ref[...],
                               preferred_element_type=jnp.float32) + b2_ref[...]


def _pool_ffn(h, batch, W1, b1, W2, b2):
    hp = jnp.zeros((NPAD, H), jnp.float32).at[:N].set(h)
    bp = jnp.full((NPAD,), -1, jnp.int32).at[:N].set(batch.astype(jnp.int32))
    bp = bp.reshape(NB, 1, BLK)
    w2p = jnp.zeros((H, 128), jnp.float32).at[:, :1].set(W2)
    b2p = jnp.zeros((1, 128), jnp.float32).at[:, :1].set(b2[None, :])
    out = pl.pallas_call(
        _pool_ffn_body,
        grid=(NB,),
        in_specs=[
            pl.BlockSpec((BLK, H), lambda i: (i, 0)),
            pl.BlockSpec((1, 1, BLK), lambda i: (i, 0, 0)),
            pl.BlockSpec((H, H), lambda i: (0, 0)),
            pl.BlockSpec((1, H), lambda i: (0, 0)),
            pl.BlockSpec((H, 128), lambda i: (0, 0)),
            pl.BlockSpec((1, 128), lambda i: (0, 0)),
        ],
        out_specs=pl.BlockSpec((B, 128), lambda i: (0, 0)),
        out_shape=jax.ShapeDtypeStruct((B, 128), jnp.float32),
        scratch_shapes=[pltpu.VMEM((B, H), jnp.float32),
                        pltpu.VMEM((B, H), jnp.float32)],
    )(hp, bp, W1, b1[None, :], w2p, b2p)
    return out[:, :1]


def kernel(x, edge_index, edge_attr, batch, Wn, bn, We0, be0, Wl, bl, Wr, br,
           Wea, att, bias, W1, b1, W2, b2):
    ei = edge_index.astype(jnp.int32)
    loop = jnp.arange(N, dtype=jnp.int32)
    row = jnp.concatenate([ei[0], loop,
                           jnp.zeros((EPP - EP,), jnp.int32)])
    col = jnp.concatenate([ei[1], loop,
                           jnp.full((EPP - EP,), N, jnp.int32)])

    h, xl, xr = _tk0(x, Wn, bn, Wl[0], bl[0], Wr[0], br[0])
    ea_mean = _tkmean(edge_attr, We0, be0)

    for l in range(DEPTH):
        eaw = _tkeaw(edge_attr, We0, be0, Wea[l], ea_mean)
        gxl, gxr = _scg(row, col, xl, xr)
        logit = _tklogit(gxl, gxr, eaw, att[l])
        pc, ac, cnt = _sk2a(row, col, logit)
        osum, den = _sk2b(xl, pc, ac, cnt)
        den = den.reshape(32, 384)[:, :W].reshape(NT, 1)
        last = l == DEPTH - 1
        if last:
            h = _tkfin(osum, den, bias[l], Wl[0], bl[0], Wr[0], br[0], True)
        else:
            h, xl, xr = _tkfin(osum, den, bias[l], Wl[l + 1], bl[l + 1],
                               Wr[l + 1], br[l + 1], False)

    return _pool_ffn(h, batch, W1, b1, W2, b2)
